# baseline (speedup 1.0000x reference)
import jax
import jax.numpy as jnp
from jax.experimental import pallas as pl

N_NODES_K = 10000
NUM_GRAPHS_K = 64


def _gru(x, h, p):
    gi = jnp.dot(x, p["W_ih"]) + p["b_ih"]
    gh = jnp.dot(h, p["W_hh"]) + p["b_hh"]
    i_r, i_z, i_n = jnp.split(gi, 3, axis=-1)
    h_r, h_z, h_n = jnp.split(gh, 3, axis=-1)
    r = jax.nn.sigmoid(i_r + h_r)
    z = jax.nn.sigmoid(i_z + h_z)
    n = jnp.tanh(i_n + r * h_n)
    return (1.0 - z) * n + z * h


def _head_kernel(out_ref, W1, b1, g1, be1, W2, b2, g2, be2, We, be_, Wo, bo, o_ref):
    out = out_ref[...]
    for W, b, g, be in ((W1, b1, g1, be1), (W2, b2, g2, be2)):
        z = jnp.dot(out, W[...], preferred_element_type=jnp.float32) + b[...]
        mu = jnp.mean(z, axis=-1, keepdims=True)
        var = jnp.mean((z - mu) ** 2, axis=-1, keepdims=True)
        z = (z - mu) / jnp.sqrt(var + 1e-5) * g[...] + be[...]
        out = jax.nn.relu(z)
    emb = jnp.dot(out, We[...], preferred_element_type=jnp.float32) + be_[...]
    o_ref[...] = jnp.dot(emb, Wo[...], preferred_element_type=jnp.float32) + bo[...]


def kernel(x, edge_index, edge_attr, batch, params):
    src = edge_index[0]
    dst = edge_index[1]
    h = x
    for lp in params["agg"]:
        hn = jnp.dot(h, lp["node"]["W"]) + lp["node"]["b"]
        e = jnp.dot(edge_attr, lp["edge"]["W"]) + lp["edge"]["b"]
        msg = jax.nn.leaky_relu(hn[src] + e, negative_slope=0.2)
        logits = jax.nn.leaky_relu(
            jnp.sum(msg * lp["a_src"], axis=-1) + jnp.sum(hn[dst] * lp["a_dst"], axis=-1),
            negative_slope=0.2)
        m = jax.ops.segment_max(logits, dst, num_segments=N_NODES_K)
        m = jnp.where(jnp.isfinite(m), m, 0.0)
        ex = jnp.exp(logits - m[dst])
        denom = jax.ops.segment_sum(ex, dst, num_segments=N_NODES_K)
        alpha = ex / (denom[dst] + 1e-16)
        agg = jax.ops.segment_sum(alpha[:, None] * msg, dst, num_segments=N_NODES_K)
        h = jax.nn.relu(_gru(agg, hn, lp["gru"]))
    out = jax.nn.relu(jax.ops.segment_sum(h, batch, num_segments=NUM_GRAPHS_K))
    for _ in range(2):
        out = jax.nn.relu(_gru(out, out, params["mol_gru"]))

    l0, l1 = params["lin"]
    args = (out,
            l0["lin"]["W"], l0["lin"]["b"], l0["gamma"], l0["beta"],
            l1["lin"]["W"], l1["lin"]["b"], l1["gamma"], l1["beta"],
            params["emb"]["W"], params["emb"]["b"],
            params["out"]["W"], params["out"]["b"])
    return pl.pallas_call(
        _head_kernel,
        out_shape=jax.ShapeDtypeStruct((NUM_GRAPHS_K, params["out"]["W"].shape[1]),
                                       jnp.float32),
    )(*args)


# speedup vs baseline: 2.0774x; 2.0774x over previous
import functools

import jax
import jax.numpy as jnp
from jax import lax
from jax.experimental import pallas as pl
from jax.experimental.pallas import tpu as pltpu
from jax.experimental.pallas import tpu_sc as plsc

N = 10000
E = 160000
D = 256
NG = 64

NC = 2
NS = 16
L = 16

CH = 48
NCHUNK = 209
E_PER_SUB = NCHUNK * CH
E_PAD = NS * E_PER_SUB
N_PAD = 10240
DST_PAD = 10200
ROWS_PER_SUB = N_PAD // NS
WCOL = 144

_MB = 5
BM = N // _MB
_EB = 88
BE = E_PAD // _EB


def _hn_body(h_ref, w_ref, b_ref, ad_ref, hn_ref, s_ref):
    hn = jnp.dot(h_ref[...], w_ref[...], preferred_element_type=jnp.float32)
    hn = hn + b_ref[...]
    hn_ref[...] = hn
    s = jnp.sum(hn * ad_ref[0:1, :], axis=-1, keepdims=True)
    s_ref[...] = jnp.broadcast_to(s, s_ref.shape)


def _hn_call(h, W, b, ad2):
    return pl.pallas_call(
        _hn_body,
        grid=(_MB,),
        in_specs=[
            pl.BlockSpec((BM, D), lambda m: (m, 0)),
            pl.BlockSpec((D, D), lambda m: (0, 0)),
            pl.BlockSpec((1, D), lambda m: (0, 0)),
            pl.BlockSpec((1, D), lambda m: (0, 0)),
        ],
        out_specs=[
            pl.BlockSpec((BM, D), lambda m: (m, 0)),
            pl.BlockSpec((BM, 128), lambda m: (m, 0)),
        ],
        out_shape=[
            jax.ShapeDtypeStruct((N, D), jnp.float32),
            jax.ShapeDtypeStruct((N, 128), jnp.float32),
        ],
    )(h, W, b, ad2)


def _e_body(ea_ref, w_ref, b_ref, e_ref):
    e = jnp.dot(ea_ref[...], w_ref[...], preferred_element_type=jnp.float32)
    e_ref[...] = e + b_ref[...]


def _e_call(edge_attr, W, b):
    de = edge_attr.shape[1]
    return pl.pallas_call(
        _e_body,
        grid=(_EB,),
        in_specs=[
            pl.BlockSpec((BE, de), lambda m: (m, 0)),
            pl.BlockSpec((de, D), lambda m: (0, 0)),
            pl.BlockSpec((1, D), lambda m: (0, 0)),
        ],
        out_specs=pl.BlockSpec((BE, D), lambda m: (m, 0)),
        out_shape=jax.ShapeDtypeStruct((E_PAD, D), jnp.float32),
    )(edge_attr, W, b)


def _sc_edge_body(hn, e, src, dst, sdst, asrc, out,
                  rows_v, e_v, out_v, src_v, dst_v, sd_v, asrc_v,
                  dots_v, w_v, acc, sem, sem2):
    c = lax.axis_index("c")
    s = lax.axis_index("s")

    pltpu.sync_copy(asrc, asrc_v)

    zero = jnp.zeros((L,), jnp.float32)

    def _zrow(i, carry):
        for j in range(WCOL // L):
            out_v[i, pl.ds(j * L, L)] = zero
        return carry

    lax.fori_loop(0, CH, _zrow, 0)
    nfull = ROWS_PER_SUB // CH
    for kk in range(nfull):
        pltpu.sync_copy(out_v, acc.at[pl.ds(s * ROWS_PER_SUB + kk * CH, CH)])
    rem = ROWS_PER_SUB - nfull * CH
    if rem:
        pltpu.sync_copy(out_v.at[pl.ds(0, rem)],
                        acc.at[pl.ds(s * ROWS_PER_SUB + nfull * CH, rem)])
    plsc.subcore_barrier()

    asrc_vals = [asrc_v[pl.ds(L * j, L)] for j in range(D // L)]
    iota_l = lax.iota(jnp.int32, L) * L
    halfoff = c * 128
    base0 = s * E_PER_SUB

    def chunk_body(k, carry):
        base = base0 + k * CH
        pltpu.sync_copy(src.at[pl.ds(base, CH)], src_v)
        pltpu.sync_copy(dst.at[pl.ds(base, CH)], dst_v)
        cp_rows = pltpu.async_copy(hn.at[src_v], rows_v, sem)
        cp_sd = pltpu.async_copy(sdst.at[dst_v], sd_v, sem2)
        pltpu.sync_copy(e.at[pl.ds(base, CH)], e_v)
        cp_rows.wait()
        cp_sd.wait()

        def group_body(g, gc):
            eb = g * L
            for i in range(L):
                acc_i = zero
                for j in range(D // L):
                    hv = rows_v[eb + i, pl.ds(L * j, L)]
                    ev = e_v[eb + i, pl.ds(L * j, L)]
                    m = hv + ev
                    m = jnp.maximum(m, 0.2 * m)
                    acc_i = acc_i + m * asrc_vals[j]
                dots_v[pl.ds(i * L, L)] = acc_i
            tot = zero
            for cc in range(L):
                tot = tot + plsc.load_gather(dots_v, [iota_l + cc])
            sd = sd_v[pl.ds(eb, L)]
            logit = tot + sd
            logit = jnp.maximum(logit, 0.2 * logit)
            w_v[pl.ds(eb, L)] = jnp.exp(logit)
            return gc

        lax.fori_loop(0, CH // L, group_body, 0)

        def edge_body(i, ec):
            wi = plsc.load_gather(w_v, [jnp.zeros((L,), jnp.int32) + i])
            for j in range(128 // L):
                hv = rows_v[i, pl.ds(halfoff + L * j, L)]
                ev = e_v[i, pl.ds(halfoff + L * j, L)]
                m = hv + ev
                m = jnp.maximum(m, 0.2 * m)
                out_v[i, pl.ds(L * j, L)] = m * wi
            out_v[i, pl.ds(128, L)] = wi
            return ec

        lax.fori_loop(0, CH, edge_body, 0)

        pltpu.sync_copy(out_v, acc.at[dst_v], add=True)
        return carry

    lax.fori_loop(0, NCHUNK, chunk_body, 0)
    plsc.subcore_barrier()

    r0 = s * ROWS_PER_SUB
    pltpu.sync_copy(acc.at[pl.ds(r0, ROWS_PER_SUB)],
                    out.at[c, pl.ds(r0, ROWS_PER_SUB)])


def _sc_edge_call(hn_b, e_b, src, dst, sdst, asrc):
    mesh = plsc.VectorSubcoreMesh(core_axis_name="c", subcore_axis_name="s")
    f = pl.kernel(
        _sc_edge_body,
        mesh=mesh,
        compiler_params=pltpu.CompilerParams(
            use_tc_tiling_on_sc=False, needs_layout_passes=False),
        out_type=jax.ShapeDtypeStruct((NC, N_PAD, WCOL), jnp.float32),
        scratch_types=[
            pltpu.VMEM((CH, D), jnp.float32),
            pltpu.VMEM((CH, D), jnp.float32),
            pltpu.VMEM((CH, WCOL), jnp.float32),
            pltpu.VMEM((CH,), jnp.int32),
            pltpu.VMEM((CH,), jnp.int32),
            pltpu.VMEM((CH,), jnp.float32),
            pltpu.VMEM((D,), jnp.float32),
            pltpu.VMEM((L * L,), jnp.float32),
            pltpu.VMEM((CH,), jnp.float32),
            pltpu.VMEM_SHARED((N_PAD, WCOL), jnp.float32),
            pltpu.SemaphoreType.DMA,
            pltpu.SemaphoreType.DMA,
        ],
    )
    return f(hn_b, e_b, src, dst, sdst, asrc)


def _gru_math(x, h, wih_ref, whh_ref, bih_ref, bhh_ref):
    gi = jnp.dot(x, wih_ref[...], preferred_element_type=jnp.float32) + bih_ref[...]
    gh = jnp.dot(h, whh_ref[...], preferred_element_type=jnp.float32) + bhh_ref[...]
    dh = h.shape[-1]
    r = jax.nn.sigmoid(gi[:, :dh] + gh[:, :dh])
    z = jax.nn.sigmoid(gi[:, dh:2 * dh] + gh[:, dh:2 * dh])
    n = jnp.tanh(gi[:, 2 * dh:] + r * gh[:, 2 * dh:])
    return (1.0 - z) * n + z * h


def _gru_body(a0_ref, a1_ref, hn_ref, wih_ref, whh_ref, bih_ref, bhh_ref, h_ref):
    a0 = a0_ref[0]
    a1 = a1_ref[0]
    den = a0[:, 128:129] + 1e-16
    num = jnp.concatenate([a0[:, :128], a1[:, :128]], axis=1)
    agg = num / den
    hn = hn_ref[...]
    h_ref[...] = jax.nn.relu(_gru_math(agg, hn, wih_ref, whh_ref, bih_ref, bhh_ref))


def _gru_call(acc, hn, gp):
    return pl.pallas_call(
        _gru_body,
        grid=(_MB,),
        in_specs=[
            pl.BlockSpec((1, BM, WCOL), lambda m: (0, m, 0)),
            pl.BlockSpec((1, BM, WCOL), lambda m: (1, m, 0)),
            pl.BlockSpec((BM, D), lambda m: (m, 0)),
            pl.BlockSpec((D, 3 * D), lambda m: (0, 0)),
            pl.BlockSpec((D, 3 * D), lambda m: (0, 0)),
            pl.BlockSpec((1, 3 * D), lambda m: (0, 0)),
            pl.BlockSpec((1, 3 * D), lambda m: (0, 0)),
        ],
        out_specs=pl.BlockSpec((BM, D), lambda m: (m, 0)),
        out_shape=jax.ShapeDtypeStruct((N, D), jnp.float32),
    )(acc, acc, hn, gp["W_ih"], gp["W_hh"],
      gp["b_ih"].reshape(1, -1), gp["b_hh"].reshape(1, -1))


def _head_body(h_ref, b_ref, wih, whh, bih, bhh,
               W1, b1, g1, be1, W2, b2, g2, be2, We, bee, Wo, bo,
               o_ref, pool_ref):
    m = pl.program_id(0)

    @pl.when(m == 0)
    def _():
        pool_ref[...] = jnp.zeros_like(pool_ref)

    onehot = (lax.broadcasted_iota(jnp.int32, (NG, BM), 0) == b_ref[0]).astype(jnp.float32)
    pool_ref[...] += jnp.dot(onehot, h_ref[...], preferred_element_type=jnp.float32,
                             precision=lax.Precision.HIGHEST)

    @pl.when(m == _MB - 1)
    def _():
        out = jax.nn.relu(pool_ref[...])
        for _ in range(2):
            out = jax.nn.relu(_gru_math(out, out, wih, whh, bih, bhh))
        for W, b, g, be in ((W1, b1, g1, be1), (W2, b2, g2, be2)):
            z = jnp.dot(out, W[...], preferred_element_type=jnp.float32) + b[...]
            mu = jnp.mean(z, axis=-1, keepdims=True)
            var = jnp.mean((z - mu) ** 2, axis=-1, keepdims=True)
            z = (z - mu) / jnp.sqrt(var + 1e-5) * g[...] + be[...]
            out = jax.nn.relu(z)
        emb = jnp.dot(out, We[...], preferred_element_type=jnp.float32) + bee[...]
        o_ref[...] = jnp.dot(emb, Wo[...], preferred_element_type=jnp.float32) + bo[...]


def _head_call(h, batch3d, params):
    l0, l1 = params["lin"]
    gp = params["mol_gru"]
    nt = params["out"]["W"].shape[1]
    d1 = l0["lin"]["W"].shape[1]
    d2 = l1["lin"]["W"].shape[1]

    def full2d(r, c):
        return pl.BlockSpec((r, c), lambda m: (0, 0))

    return pl.pallas_call(
        _head_body,
        grid=(_MB,),
        in_specs=[
            pl.BlockSpec((BM, D), lambda m: (m, 0)),
            pl.BlockSpec((1, 1, BM), lambda m: (m, 0, 0)),
            full2d(D, 3 * D), full2d(D, 3 * D), full2d(1, 3 * D), full2d(1, 3 * D),
            full2d(D, d1), full2d(1, d1), full2d(1, d1), full2d(1, d1),
            full2d(d1, d2), full2d(1, d2), full2d(1, d2), full2d(1, d2),
            full2d(d2, d2), full2d(1, d2),
            full2d(d2, nt), full2d(1, nt),
        ],
        out_specs=pl.BlockSpec((NG, nt), lambda m: (0, 0)),
        out_shape=jax.ShapeDtypeStruct((NG, nt), jnp.float32),
        scratch_shapes=[pltpu.VMEM((NG, D), jnp.float32)],
    )(h, batch3d,
      gp["W_ih"], gp["W_hh"], gp["b_ih"].reshape(1, -1), gp["b_hh"].reshape(1, -1),
      l0["lin"]["W"], l0["lin"]["b"].reshape(1, -1),
      l0["gamma"].reshape(1, -1), l0["beta"].reshape(1, -1),
      l1["lin"]["W"], l1["lin"]["b"].reshape(1, -1),
      l1["gamma"].reshape(1, -1), l1["beta"].reshape(1, -1),
      params["emb"]["W"], params["emb"]["b"].reshape(1, -1),
      params["out"]["W"], params["out"]["b"].reshape(1, -1))


def kernel(x, edge_index, edge_attr, batch, params):
    npad = E_PAD - E
    src = jnp.pad(edge_index[0].astype(jnp.int32), (0, npad))
    dst = jnp.pad(edge_index[1].astype(jnp.int32), (0, npad),
                  constant_values=DST_PAD)
    ea_pad = jnp.pad(edge_attr, ((0, npad), (0, 0)))
    batch3d = batch.astype(jnp.int32).reshape(_MB, 1, BM)

    h = x
    for lp in params["agg"]:
        ad2 = lp["a_dst"].reshape(1, -1)
        hn, s2d = _hn_call(h, lp["node"]["W"], lp["node"]["b"].reshape(1, -1), ad2)
        sdst = jnp.pad(s2d[:, 0], (0, N_PAD - N))
        e = _e_call(ea_pad, lp["edge"]["W"], lp["edge"]["b"].reshape(1, -1))
        acc = _sc_edge_call(hn, e, src, dst, sdst, lp["a_src"])
        h = _gru_call(acc, hn, lp["gru"])

    return _head_call(h, batch3d, params)


# speedup vs baseline: 2.4452x; 1.1771x over previous
import functools

import jax
import jax.numpy as jnp
from jax import lax
from jax.experimental import pallas as pl
from jax.experimental.pallas import tpu as pltpu
from jax.experimental.pallas import tpu_sc as plsc

N = 10000
E = 160000
D = 256
NG = 64

NC = 2
NS = 16
L = 16

CH = 32
NCHUNK = 316
E_PER_SUB = NCHUNK * CH
E_PAD = NS * E_PER_SUB
N_PAD = 10016
DST_PAD = 10008
ROWS_PER_SUB = N_PAD // NS
WCOL = 144

_MB = 5
BM = N // _MB
_EB = 79
BE = E_PAD // _EB


def _hn_body(h_ref, w_ref, b_ref, ad_ref, hn_ref, s_ref):
    hn = jnp.dot(h_ref[...], w_ref[...], preferred_element_type=jnp.float32)
    hn = hn + b_ref[...]
    hn_ref[...] = hn
    s = jnp.sum(hn * ad_ref[0:1, :], axis=-1, keepdims=True)
    s_ref[...] = jnp.broadcast_to(s, s_ref.shape)


def _hn_call(h, W, b, ad2):
    return pl.pallas_call(
        _hn_body,
        grid=(_MB,),
        in_specs=[
            pl.BlockSpec((BM, D), lambda m: (m, 0)),
            pl.BlockSpec((D, D), lambda m: (0, 0)),
            pl.BlockSpec((1, D), lambda m: (0, 0)),
            pl.BlockSpec((1, D), lambda m: (0, 0)),
        ],
        out_specs=[
            pl.BlockSpec((BM, D), lambda m: (m, 0)),
            pl.BlockSpec((BM, 128), lambda m: (m, 0)),
        ],
        out_shape=[
            jax.ShapeDtypeStruct((N, D), jnp.float32),
            jax.ShapeDtypeStruct((N, 128), jnp.float32),
        ],
    )(h, W, b, ad2)


def _e_body(ea_ref, w_ref, b_ref, e_ref):
    e = jnp.dot(ea_ref[...], w_ref[...], preferred_element_type=jnp.float32)
    e_ref[...] = e + b_ref[...]


def _e_call(edge_attr, W, b):
    de = edge_attr.shape[1]
    return pl.pallas_call(
        _e_body,
        grid=(_EB,),
        in_specs=[
            pl.BlockSpec((BE, de), lambda m: (m, 0)),
            pl.BlockSpec((de, D), lambda m: (0, 0)),
            pl.BlockSpec((1, D), lambda m: (0, 0)),
        ],
        out_specs=pl.BlockSpec((BE, D), lambda m: (m, 0)),
        out_shape=jax.ShapeDtypeStruct((E_PAD, D), jnp.float32),
    )(edge_attr, W, b)


def _sc_edge_body(hn, e, src, dst, sdst, asrc, out,
                  rows_v0, rows_v1, e_v0, e_v1, out_v,
                  src_v0, src_v1, dst_v0, dst_v1, sd_v0, sd_v1,
                  asrc_v, dots_v, w_v, acc,
                  sem_i0, sem_i1, sem_r0, sem_r1, sem_e0, sem_e1,
                  sem_s0, sem_s1):
    c = lax.axis_index("c")
    s = lax.axis_index("s")

    rows_b = (rows_v0, rows_v1)
    e_b = (e_v0, e_v1)
    src_b = (src_v0, src_v1)
    dst_b = (dst_v0, dst_v1)
    sd_b = (sd_v0, sd_v1)
    sem_i = (sem_i0, sem_i1)
    sem_r = (sem_r0, sem_r1)
    sem_e = (sem_e0, sem_e1)
    sem_s = (sem_s0, sem_s1)

    pltpu.sync_copy(asrc, asrc_v)

    zero = jnp.zeros((L,), jnp.float32)

    def _zrow(i, carry):
        for j in range(WCOL // L):
            out_v[i, pl.ds(j * L, L)] = zero
        return carry

    lax.fori_loop(0, CH, _zrow, 0)
    nfull = ROWS_PER_SUB // CH
    for kk in range(nfull):
        pltpu.sync_copy(out_v, acc.at[pl.ds(s * ROWS_PER_SUB + kk * CH, CH)])
    rem = ROWS_PER_SUB - nfull * CH
    if rem:
        pltpu.sync_copy(out_v.at[pl.ds(0, rem)],
                        acc.at[pl.ds(s * ROWS_PER_SUB + nfull * CH, rem)])
    plsc.subcore_barrier()

    asrc_vals = [asrc_v[pl.ds(L * j, L)] for j in range(D // L)]
    iota_l = lax.iota(jnp.int32, L) * L
    halfoff = c * 128
    base0 = s * E_PER_SUB

    def idx_copy_start(k, b):
        base = base0 + k * CH
        pltpu.async_copy(src.at[pl.ds(base, CH)], src_b[b], sem_i[b])
        pltpu.async_copy(dst.at[pl.ds(base, CH)], dst_b[b], sem_i[b])

    def idx_copy_wait(k, b):
        base = base0 + k * CH
        pltpu.make_async_copy(src.at[pl.ds(base, CH)], src_b[b], sem_i[b]).wait()
        pltpu.make_async_copy(dst.at[pl.ds(base, CH)], dst_b[b], sem_i[b]).wait()

    def gather_start(k, b):
        base = base0 + k * CH
        pltpu.async_copy(hn.at[src_b[b]], rows_b[b], sem_r[b])
        pltpu.async_copy(e.at[pl.ds(base, CH)], e_b[b], sem_e[b])
        pltpu.async_copy(sdst.at[dst_b[b]], sd_b[b], sem_s[b])

    def gather_wait(k, b):
        base = base0 + k * CH
        pltpu.make_async_copy(hn.at[src_b[b]], rows_b[b], sem_r[b]).wait()
        pltpu.make_async_copy(e.at[pl.ds(base, CH)], e_b[b], sem_e[b]).wait()
        pltpu.make_async_copy(sdst.at[dst_b[b]], sd_b[b], sem_s[b]).wait()

    def compute_chunk(b):
        rows_v = rows_b[b]
        e_v = e_b[b]
        sd_v = sd_b[b]

        def group_body(g, gc):
            eb = g * L
            for i in range(L):
                acc_i = zero
                for j in range(D // L):
                    hv = rows_v[eb + i, pl.ds(L * j, L)]
                    ev = e_v[eb + i, pl.ds(L * j, L)]
                    m = hv + ev
                    m = jnp.maximum(m, 0.2 * m)
                    acc_i = acc_i + m * asrc_vals[j]
                dots_v[pl.ds(i * L, L)] = acc_i
            tot = zero
            for cc in range(L):
                tot = tot + plsc.load_gather(dots_v, [iota_l + cc])
            sd = sd_v[pl.ds(eb, L)]
            logit = tot + sd
            logit = jnp.maximum(logit, 0.2 * logit)
            w_v[pl.ds(eb, L)] = jnp.exp(logit)
            return gc

        lax.fori_loop(0, CH // L, group_body, 0)

        def edge_body(i, ec):
            wi = plsc.load_gather(w_v, [jnp.zeros((L,), jnp.int32) + i])
            for j in range(128 // L):
                hv = rows_v[i, pl.ds(halfoff + L * j, L)]
                ev = e_v[i, pl.ds(halfoff + L * j, L)]
                m = hv + ev
                m = jnp.maximum(m, 0.2 * m)
                out_v[i, pl.ds(L * j, L)] = m * wi
            out_v[i, pl.ds(128, L)] = wi
            return ec

        lax.fori_loop(0, CH, edge_body, 0)

    idx_copy_start(0, 0)
    idx_copy_wait(0, 0)
    gather_start(0, 0)
    idx_copy_start(1, 1)

    def chunk_pair(t, carry):
        for b in range(2):
            k = 2 * t + b
            nb = 1 - b

            @pl.when(k + 1 < NCHUNK)
            def _():
                idx_copy_wait(k + 1, nb)
                gather_start(k + 1, nb)

            gather_wait(k, b)
            compute_chunk(b)
            pltpu.sync_copy(out_v, acc.at[dst_b[b]], add=True)

            @pl.when(k + 2 < NCHUNK)
            def _():
                idx_copy_start(k + 2, b)
        return carry

    lax.fori_loop(0, NCHUNK // 2, chunk_pair, 0)
    plsc.subcore_barrier()

    r0 = s * ROWS_PER_SUB
    pltpu.sync_copy(acc.at[pl.ds(r0, ROWS_PER_SUB)],
                    out.at[c, pl.ds(r0, ROWS_PER_SUB)])


def _sc_edge_call(hn_b, e_b, src, dst, sdst, asrc):
    mesh = plsc.VectorSubcoreMesh(core_axis_name="c", subcore_axis_name="s")
    f = pl.kernel(
        _sc_edge_body,
        mesh=mesh,
        compiler_params=pltpu.CompilerParams(
            use_tc_tiling_on_sc=False, needs_layout_passes=False),
        out_type=jax.ShapeDtypeStruct((NC, N_PAD, WCOL), jnp.float32),
        scratch_types=[
            pltpu.VMEM((CH, D), jnp.float32),
            pltpu.VMEM((CH, D), jnp.float32),
            pltpu.VMEM((CH, D), jnp.float32),
            pltpu.VMEM((CH, D), jnp.float32),
            pltpu.VMEM((CH, WCOL), jnp.float32),
            pltpu.VMEM((CH,), jnp.int32),
            pltpu.VMEM((CH,), jnp.int32),
            pltpu.VMEM((CH,), jnp.int32),
            pltpu.VMEM((CH,), jnp.int32),
            pltpu.VMEM((CH,), jnp.float32),
            pltpu.VMEM((CH,), jnp.float32),
            pltpu.VMEM((D,), jnp.float32),
            pltpu.VMEM((L * L,), jnp.float32),
            pltpu.VMEM((CH,), jnp.float32),
            pltpu.VMEM_SHARED((N_PAD, WCOL), jnp.float32),
            pltpu.SemaphoreType.DMA,
            pltpu.SemaphoreType.DMA,
            pltpu.SemaphoreType.DMA,
            pltpu.SemaphoreType.DMA,
            pltpu.SemaphoreType.DMA,
            pltpu.SemaphoreType.DMA,
            pltpu.SemaphoreType.DMA,
            pltpu.SemaphoreType.DMA,
        ],
    )
    return f(hn_b, e_b, src, dst, sdst, asrc)


def _gru_math(x, h, wih_ref, whh_ref, bih_ref, bhh_ref):
    gi = jnp.dot(x, wih_ref[...], preferred_element_type=jnp.float32) + bih_ref[...]
    gh = jnp.dot(h, whh_ref[...], preferred_element_type=jnp.float32) + bhh_ref[...]
    dh = h.shape[-1]
    r = jax.nn.sigmoid(gi[:, :dh] + gh[:, :dh])
    z = jax.nn.sigmoid(gi[:, dh:2 * dh] + gh[:, dh:2 * dh])
    n = jnp.tanh(gi[:, 2 * dh:] + r * gh[:, 2 * dh:])
    return (1.0 - z) * n + z * h


def _gru_body(a0_ref, a1_ref, hn_ref, wih_ref, whh_ref, bih_ref, bhh_ref, h_ref):
    a0 = a0_ref[0]
    a1 = a1_ref[0]
    den = a0[:, 128:129] + 1e-16
    num = jnp.concatenate([a0[:, :128], a1[:, :128]], axis=1)
    agg = num / den
    hn = hn_ref[...]
    h_ref[...] = jax.nn.relu(_gru_math(agg, hn, wih_ref, whh_ref, bih_ref, bhh_ref))


def _gru_call(acc, hn, gp):
    return pl.pallas_call(
        _gru_body,
        grid=(_MB,),
        in_specs=[
            pl.BlockSpec((1, BM, WCOL), lambda m: (0, m, 0)),
            pl.BlockSpec((1, BM, WCOL), lambda m: (1, m, 0)),
            pl.BlockSpec((BM, D), lambda m: (m, 0)),
            pl.BlockSpec((D, 3 * D), lambda m: (0, 0)),
            pl.BlockSpec((D, 3 * D), lambda m: (0, 0)),
            pl.BlockSpec((1, 3 * D), lambda m: (0, 0)),
            pl.BlockSpec((1, 3 * D), lambda m: (0, 0)),
        ],
        out_specs=pl.BlockSpec((BM, D), lambda m: (m, 0)),
        out_shape=jax.ShapeDtypeStruct((N, D), jnp.float32),
    )(acc, acc, hn, gp["W_ih"], gp["W_hh"],
      gp["b_ih"].reshape(1, -1), gp["b_hh"].reshape(1, -1))


def _head_body(h_ref, b_ref, wih, whh, bih, bhh,
               W1, b1, g1, be1, W2, b2, g2, be2, We, bee, Wo, bo,
               o_ref, pool_ref):
    m = pl.program_id(0)

    @pl.when(m == 0)
    def _():
        pool_ref[...] = jnp.zeros_like(pool_ref)

    onehot = (lax.broadcasted_iota(jnp.int32, (NG, BM), 0) == b_ref[0]).astype(jnp.float32)
    pool_ref[...] += jnp.dot(onehot, h_ref[...], preferred_element_type=jnp.float32,
                             precision=lax.Precision.HIGHEST)

    @pl.when(m == _MB - 1)
    def _():
        out = jax.nn.relu(pool_ref[...])
        for _ in range(2):
            out = jax.nn.relu(_gru_math(out, out, wih, whh, bih, bhh))
        for W, b, g, be in ((W1, b1, g1, be1), (W2, b2, g2, be2)):
            z = jnp.dot(out, W[...], preferred_element_type=jnp.float32) + b[...]
            mu = jnp.mean(z, axis=-1, keepdims=True)
            var = jnp.mean((z - mu) ** 2, axis=-1, keepdims=True)
            z = (z - mu) / jnp.sqrt(var + 1e-5) * g[...] + be[...]
            out = jax.nn.relu(z)
        emb = jnp.dot(out, We[...], preferred_element_type=jnp.float32) + bee[...]
        o_ref[...] = jnp.dot(emb, Wo[...], preferred_element_type=jnp.float32) + bo[...]


def _head_call(h, batch3d, params):
    l0, l1 = params["lin"]
    gp = params["mol_gru"]
    nt = params["out"]["W"].shape[1]
    d1 = l0["lin"]["W"].shape[1]
    d2 = l1["lin"]["W"].shape[1]

    def full2d(r, c):
        return pl.BlockSpec((r, c), lambda m: (0, 0))

    return pl.pallas_call(
        _head_body,
        grid=(_MB,),
        in_specs=[
            pl.BlockSpec((BM, D), lambda m: (m, 0)),
            pl.BlockSpec((1, 1, BM), lambda m: (m, 0, 0)),
            full2d(D, 3 * D), full2d(D, 3 * D), full2d(1, 3 * D), full2d(1, 3 * D),
            full2d(D, d1), full2d(1, d1), full2d(1, d1), full2d(1, d1),
            full2d(d1, d2), full2d(1, d2), full2d(1, d2), full2d(1, d2),
            full2d(d2, d2), full2d(1, d2),
            full2d(d2, nt), full2d(1, nt),
        ],
        out_specs=pl.BlockSpec((NG, nt), lambda m: (0, 0)),
        out_shape=jax.ShapeDtypeStruct((NG, nt), jnp.float32),
        scratch_shapes=[pltpu.VMEM((NG, D), jnp.float32)],
    )(h, batch3d,
      gp["W_ih"], gp["W_hh"], gp["b_ih"].reshape(1, -1), gp["b_hh"].reshape(1, -1),
      l0["lin"]["W"], l0["lin"]["b"].reshape(1, -1),
      l0["gamma"].reshape(1, -1), l0["beta"].reshape(1, -1),
      l1["lin"]["W"], l1["lin"]["b"].reshape(1, -1),
      l1["gamma"].reshape(1, -1), l1["beta"].reshape(1, -1),
      params["emb"]["W"], params["emb"]["b"].reshape(1, -1),
      params["out"]["W"], params["out"]["b"].reshape(1, -1))


def kernel(x, edge_index, edge_attr, batch, params):
    npad = E_PAD - E
    src = jnp.pad(edge_index[0].astype(jnp.int32), (0, npad))
    dst = jnp.pad(edge_index[1].astype(jnp.int32), (0, npad),
                  constant_values=DST_PAD)
    ea_pad = jnp.pad(edge_attr, ((0, npad), (0, 0)))
    batch3d = batch.astype(jnp.int32).reshape(_MB, 1, BM)

    h = x
    for lp in params["agg"]:
        ad2 = lp["a_dst"].reshape(1, -1)
        hn, s2d = _hn_call(h, lp["node"]["W"], lp["node"]["b"].reshape(1, -1), ad2)
        sdst = jnp.pad(s2d[:, 0], (0, N_PAD - N))
        e = _e_call(ea_pad, lp["edge"]["W"], lp["edge"]["b"].reshape(1, -1))
        acc = _sc_edge_call(hn, e, src, dst, sdst, lp["a_src"])
        h = _gru_call(acc, hn, lp["gru"])

    return _head_call(h, batch3d, params)


# speedup vs baseline: 2.4476x; 1.0010x over previous
import functools

import jax
import jax.numpy as jnp
from jax import lax
from jax.experimental import pallas as pl
from jax.experimental.pallas import tpu as pltpu
from jax.experimental.pallas import tpu_sc as plsc

N = 10000
E = 160000
D = 256
NG = 64

NC = 2
NS = 16
L = 16

CH = 32
NCHUNK_BASE = 312
N_PAD = 10016
ROWS_PER_SUB = N_PAD // NS

_MB = 5
BM = N // _MB
_EB = 80
BE = E // _EB


def _hn_body(h_ref, w_ref, b_ref, ad_ref, hn_ref, s_ref):
    hn = jnp.dot(h_ref[...], w_ref[...], preferred_element_type=jnp.float32)
    hn = hn + b_ref[...]
    hn_ref[...] = hn
    s = jnp.sum(hn * ad_ref[0:1, :], axis=-1, keepdims=True)
    s_ref[...] = jnp.broadcast_to(s, s_ref.shape)


def _hn_call(h, W, b, ad2):
    return pl.pallas_call(
        _hn_body,
        grid=(_MB,),
        in_specs=[
            pl.BlockSpec((BM, D), lambda m: (m, 0)),
            pl.BlockSpec((D, D), lambda m: (0, 0)),
            pl.BlockSpec((1, D), lambda m: (0, 0)),
            pl.BlockSpec((1, D), lambda m: (0, 0)),
        ],
        out_specs=[
            pl.BlockSpec((BM, D), lambda m: (m, 0)),
            pl.BlockSpec((BM, 128), lambda m: (m, 0)),
        ],
        out_shape=[
            jax.ShapeDtypeStruct((N, D), jnp.float32),
            jax.ShapeDtypeStruct((N, 128), jnp.float32),
        ],
    )(h, W, b, ad2)


def _e_body(ea_ref, w_ref, b_ref, e_ref):
    e = jnp.dot(ea_ref[...], w_ref[...], preferred_element_type=jnp.float32)
    e_ref[...] = e + b_ref[...]


def _e_call(edge_attr, W, b):
    de = edge_attr.shape[1]
    return pl.pallas_call(
        _e_body,
        grid=(_EB,),
        in_specs=[
            pl.BlockSpec((BE, de), lambda m: (m, 0)),
            pl.BlockSpec((de, D), lambda m: (0, 0)),
            pl.BlockSpec((1, D), lambda m: (0, 0)),
        ],
        out_specs=pl.BlockSpec((BE, D), lambda m: (m, 0)),
        out_shape=jax.ShapeDtypeStruct((E, D), jnp.float32),
    )(edge_attr, W, b)


def _sc_edge_body(hn, e, src, dst, sdst, asrc, num, den,
                  rows_v0, rows_v1, e_v0, e_v1, out_v, den_v,
                  src_v0, src_v1, dst_v0, dst_v1, sd_v0, sd_v1,
                  asrc_v, dots_v, w_v, acc, dacc,
                  sem_i0, sem_i1, sem_r0, sem_r1, sem_e0, sem_e1,
                  sem_s0, sem_s1):
    c = lax.axis_index("c")
    s = lax.axis_index("s")

    rows_b = (rows_v0, rows_v1)
    e_b = (e_v0, e_v1)
    src_b = (src_v0, src_v1)
    dst_b = (dst_v0, dst_v1)
    sd_b = (sd_v0, sd_v1)
    sem_i = (sem_i0, sem_i1)
    sem_r = (sem_r0, sem_r1)
    sem_e = (sem_e0, sem_e1)
    sem_s = (sem_s0, sem_s1)

    pltpu.sync_copy(asrc, asrc_v)

    zero = jnp.zeros((L,), jnp.float32)

    def _zrow(i, carry):
        for j in range(128 // L):
            out_v[i, pl.ds(j * L, L)] = zero
        den_v[i, :] = zero
        return carry

    lax.fori_loop(0, CH, _zrow, 0)
    nfull = ROWS_PER_SUB // CH
    for kk in range(nfull):
        r0 = s * ROWS_PER_SUB + kk * CH
        pltpu.sync_copy(out_v, acc.at[pl.ds(r0, CH)])
        pltpu.sync_copy(den_v, dacc.at[pl.ds(r0, CH)])
    rem = ROWS_PER_SUB - nfull * CH
    if rem:
        r0 = s * ROWS_PER_SUB + nfull * CH
        pltpu.sync_copy(out_v.at[pl.ds(0, rem)], acc.at[pl.ds(r0, rem)])
        pltpu.sync_copy(den_v.at[pl.ds(0, rem)], dacc.at[pl.ds(r0, rem)])
    plsc.subcore_barrier()

    asrc_vals = [asrc_v[pl.ds(L * j, L)] for j in range(D // L)]
    iota_l = lax.iota(jnp.int32, L) * L
    halfoff = c * 128
    nchunk = NCHUNK_BASE + jnp.where(s < 8, 1, 0)
    base0 = (s * NCHUNK_BASE + jnp.minimum(s, 8)) * CH

    def idx_copy_start(k, b):
        base = base0 + k * CH
        pltpu.async_copy(src.at[pl.ds(base, CH)], src_b[b], sem_i[b])
        pltpu.async_copy(dst.at[pl.ds(base, CH)], dst_b[b], sem_i[b])

    def idx_copy_wait(k, b):
        base = base0 + k * CH
        pltpu.make_async_copy(src.at[pl.ds(base, CH)], src_b[b], sem_i[b]).wait()
        pltpu.make_async_copy(dst.at[pl.ds(base, CH)], dst_b[b], sem_i[b]).wait()

    def gather_start(k, b):
        base = base0 + k * CH
        pltpu.async_copy(hn.at[src_b[b]], rows_b[b], sem_r[b])
        pltpu.async_copy(e.at[pl.ds(base, CH)], e_b[b], sem_e[b])
        pltpu.async_copy(sdst.at[dst_b[b]], sd_b[b], sem_s[b])

    def gather_wait(k, b):
        base = base0 + k * CH
        pltpu.make_async_copy(hn.at[src_b[b]], rows_b[b], sem_r[b]).wait()
        pltpu.make_async_copy(e.at[pl.ds(base, CH)], e_b[b], sem_e[b]).wait()
        pltpu.make_async_copy(sdst.at[dst_b[b]], sd_b[b], sem_s[b]).wait()

    def compute_chunk(b):
        rows_v = rows_b[b]
        e_v = e_b[b]
        sd_v = sd_b[b]

        def group_body(g, gc):
            eb = g * L
            for i in range(L):
                acc_i = zero
                for j in range(D // L):
                    hv = rows_v[eb + i, pl.ds(L * j, L)]
                    ev = e_v[eb + i, pl.ds(L * j, L)]
                    m = hv + ev
                    m = jnp.maximum(m, 0.2 * m)
                    acc_i = acc_i + m * asrc_vals[j]
                dots_v[pl.ds(i * L, L)] = acc_i
            tot = zero
            for cc in range(L):
                tot = tot + plsc.load_gather(dots_v, [iota_l + cc])
            sd = sd_v[pl.ds(eb, L)]
            logit = tot + sd
            logit = jnp.maximum(logit, 0.2 * logit)
            w_v[pl.ds(eb, L)] = jnp.exp(logit)
            return gc

        lax.fori_loop(0, CH // L, group_body, 0)

        def edge_body(i, ec):
            wi = plsc.load_gather(w_v, [jnp.zeros((L,), jnp.int32) + i])
            for j in range(128 // L):
                hv = rows_v[i, pl.ds(halfoff + L * j, L)]
                ev = e_v[i, pl.ds(halfoff + L * j, L)]
                m = hv + ev
                m = jnp.maximum(m, 0.2 * m)
                out_v[i, pl.ds(L * j, L)] = m * wi
            den_v[i, :] = wi
            return ec

        lax.fori_loop(0, CH, edge_body, 0)

    idx_copy_start(0, 0)
    idx_copy_wait(0, 0)
    gather_start(0, 0)
    idx_copy_start(1, 1)

    def chunk_pair(t, carry):
        for b in range(2):
            k = 2 * t + b
            nb = 1 - b

            @pl.when(k + 1 < nchunk)
            def _():
                idx_copy_wait(k + 1, nb)
                gather_start(k + 1, nb)

            @pl.when(k < nchunk)
            def _():
                gather_wait(k, b)
                compute_chunk(b)
                pltpu.sync_copy(out_v, acc.at[dst_b[b]], add=True)
                pltpu.sync_copy(den_v, dacc.at[dst_b[b]], add=True)

            @pl.when(k + 2 < nchunk)
            def _():
                idx_copy_start(k + 2, b)
        return carry

    lax.fori_loop(0, (NCHUNK_BASE + 2) // 2, chunk_pair, 0)
    plsc.subcore_barrier()

    r0 = s * ROWS_PER_SUB
    pltpu.sync_copy(acc.at[pl.ds(r0, ROWS_PER_SUB)],
                    num.at[c, pl.ds(r0, ROWS_PER_SUB)])
    pltpu.sync_copy(dacc.at[pl.ds(r0, ROWS_PER_SUB)],
                    den.at[c, pl.ds(r0, ROWS_PER_SUB)])


def _sc_edge_call(hn_b, e_b, src, dst, sdst, asrc):
    mesh = plsc.VectorSubcoreMesh(core_axis_name="c", subcore_axis_name="s")
    f = pl.kernel(
        _sc_edge_body,
        mesh=mesh,
        compiler_params=pltpu.CompilerParams(
            use_tc_tiling_on_sc=False, needs_layout_passes=False),
        out_type=[
            jax.ShapeDtypeStruct((NC, N_PAD, 128), jnp.float32),
            jax.ShapeDtypeStruct((NC, N_PAD, L), jnp.float32),
        ],
        scratch_types=[
            pltpu.VMEM((CH, D), jnp.float32),
            pltpu.VMEM((CH, D), jnp.float32),
            pltpu.VMEM((CH, D), jnp.float32),
            pltpu.VMEM((CH, D), jnp.float32),
            pltpu.VMEM((CH, 128), jnp.float32),
            pltpu.VMEM((CH, L), jnp.float32),
            pltpu.VMEM((CH,), jnp.int32),
            pltpu.VMEM((CH,), jnp.int32),
            pltpu.VMEM((CH,), jnp.int32),
            pltpu.VMEM((CH,), jnp.int32),
            pltpu.VMEM((CH,), jnp.float32),
            pltpu.VMEM((CH,), jnp.float32),
            pltpu.VMEM((D,), jnp.float32),
            pltpu.VMEM((L * L,), jnp.float32),
            pltpu.VMEM((CH,), jnp.float32),
            pltpu.VMEM_SHARED((N_PAD, 128), jnp.float32),
            pltpu.VMEM_SHARED((N_PAD, L), jnp.float32),
            pltpu.SemaphoreType.DMA,
            pltpu.SemaphoreType.DMA,
            pltpu.SemaphoreType.DMA,
            pltpu.SemaphoreType.DMA,
            pltpu.SemaphoreType.DMA,
            pltpu.SemaphoreType.DMA,
            pltpu.SemaphoreType.DMA,
            pltpu.SemaphoreType.DMA,
        ],
    )
    return f(hn_b, e_b, src, dst, sdst, asrc)


def _gru_math(x, h, wih_ref, whh_ref, bih_ref, bhh_ref):
    gi = jnp.dot(x, wih_ref[...], preferred_element_type=jnp.float32) + bih_ref[...]
    gh = jnp.dot(h, whh_ref[...], preferred_element_type=jnp.float32) + bhh_ref[...]
    dh = h.shape[-1]
    r = jax.nn.sigmoid(gi[:, :dh] + gh[:, :dh])
    z = jax.nn.sigmoid(gi[:, dh:2 * dh] + gh[:, dh:2 * dh])
    n = jnp.tanh(gi[:, 2 * dh:] + r * gh[:, 2 * dh:])
    return (1.0 - z) * n + z * h


def _gru_body(a0_ref, a1_ref, dn_ref, hn_ref, wih_ref, whh_ref, bih_ref,
              bhh_ref, h_ref):
    den = dn_ref[0][:, 0:1] + 1e-16
    num = jnp.concatenate([a0_ref[0], a1_ref[0]], axis=1)
    agg = num / den
    hn = hn_ref[...]
    h_ref[...] = jax.nn.relu(_gru_math(agg, hn, wih_ref, whh_ref, bih_ref, bhh_ref))


def _gru_call(num, den, hn, gp):
    return pl.pallas_call(
        _gru_body,
        grid=(_MB,),
        in_specs=[
            pl.BlockSpec((1, BM, 128), lambda m: (0, m, 0)),
            pl.BlockSpec((1, BM, 128), lambda m: (1, m, 0)),
            pl.BlockSpec((1, BM, L), lambda m: (0, m, 0)),
            pl.BlockSpec((BM, D), lambda m: (m, 0)),
            pl.BlockSpec((D, 3 * D), lambda m: (0, 0)),
            pl.BlockSpec((D, 3 * D), lambda m: (0, 0)),
            pl.BlockSpec((1, 3 * D), lambda m: (0, 0)),
            pl.BlockSpec((1, 3 * D), lambda m: (0, 0)),
        ],
        out_specs=pl.BlockSpec((BM, D), lambda m: (m, 0)),
        out_shape=jax.ShapeDtypeStruct((N, D), jnp.float32),
    )(num, num, den, hn, gp["W_ih"], gp["W_hh"],
      gp["b_ih"].reshape(1, -1), gp["b_hh"].reshape(1, -1))


def _head_body(h_ref, b_ref, wih, whh, bih, bhh,
               W1, b1, g1, be1, W2, b2, g2, be2, We, bee, Wo, bo,
               o_ref, pool_ref):
    m = pl.program_id(0)

    @pl.when(m == 0)
    def _():
        pool_ref[...] = jnp.zeros_like(pool_ref)

    onehot = (lax.broadcasted_iota(jnp.int32, (NG, BM), 0) == b_ref[0]).astype(jnp.float32)
    pool_ref[...] += jnp.dot(onehot, h_ref[...], preferred_element_type=jnp.float32,
                             precision=lax.Precision.HIGHEST)

    @pl.when(m == _MB - 1)
    def _():
        out = jax.nn.relu(pool_ref[...])
        for _ in range(2):
            out = jax.nn.relu(_gru_math(out, out, wih, whh, bih, bhh))
        for W, b, g, be in ((W1, b1, g1, be1), (W2, b2, g2, be2)):
            z = jnp.dot(out, W[...], preferred_element_type=jnp.float32) + b[...]
            mu = jnp.mean(z, axis=-1, keepdims=True)
            var = jnp.mean((z - mu) ** 2, axis=-1, keepdims=True)
            z = (z - mu) / jnp.sqrt(var + 1e-5) * g[...] + be[...]
            out = jax.nn.relu(z)
        emb = jnp.dot(out, We[...], preferred_element_type=jnp.float32) + bee[...]
        o_ref[...] = jnp.dot(emb, Wo[...], preferred_element_type=jnp.float32) + bo[...]


def _head_call(h, batch3d, params):
    l0, l1 = params["lin"]
    gp = params["mol_gru"]
    nt = params["out"]["W"].shape[1]
    d1 = l0["lin"]["W"].shape[1]
    d2 = l1["lin"]["W"].shape[1]

    def full2d(r, c):
        return pl.BlockSpec((r, c), lambda m: (0, 0))

    return pl.pallas_call(
        _head_body,
        grid=(_MB,),
        in_specs=[
            pl.BlockSpec((BM, D), lambda m: (m, 0)),
            pl.BlockSpec((1, 1, BM), lambda m: (m, 0, 0)),
            full2d(D, 3 * D), full2d(D, 3 * D), full2d(1, 3 * D), full2d(1, 3 * D),
            full2d(D, d1), full2d(1, d1), full2d(1, d1), full2d(1, d1),
            full2d(d1, d2), full2d(1, d2), full2d(1, d2), full2d(1, d2),
            full2d(d2, d2), full2d(1, d2),
            full2d(d2, nt), full2d(1, nt),
        ],
        out_specs=pl.BlockSpec((NG, nt), lambda m: (0, 0)),
        out_shape=jax.ShapeDtypeStruct((NG, nt), jnp.float32),
        scratch_shapes=[pltpu.VMEM((NG, D), jnp.float32)],
    )(h, batch3d,
      gp["W_ih"], gp["W_hh"], gp["b_ih"].reshape(1, -1), gp["b_hh"].reshape(1, -1),
      l0["lin"]["W"], l0["lin"]["b"].reshape(1, -1),
      l0["gamma"].reshape(1, -1), l0["beta"].reshape(1, -1),
      l1["lin"]["W"], l1["lin"]["b"].reshape(1, -1),
      l1["gamma"].reshape(1, -1), l1["beta"].reshape(1, -1),
      params["emb"]["W"], params["emb"]["b"].reshape(1, -1),
      params["out"]["W"], params["out"]["b"].reshape(1, -1))


def kernel(x, edge_index, edge_attr, batch, params):
    src = edge_index[0].astype(jnp.int32)
    dst = edge_index[1].astype(jnp.int32)
    batch3d = batch.astype(jnp.int32).reshape(_MB, 1, BM)

    h = x
    for lp in params["agg"]:
        ad2 = lp["a_dst"].reshape(1, -1)
        hn, s2d = _hn_call(h, lp["node"]["W"], lp["node"]["b"].reshape(1, -1), ad2)
        sdst = s2d[:, 0]
        e = _e_call(edge_attr, lp["edge"]["W"], lp["edge"]["b"].reshape(1, -1))
        num, den = _sc_edge_call(hn, e, src, dst, sdst, lp["a_src"])
        h = _gru_call(num, den, hn, lp["gru"])

    return _head_call(h, batch3d, params)


# speedup vs baseline: 2.5824x; 1.0551x over previous
import functools

import jax
import jax.numpy as jnp
from jax import lax
from jax.experimental import pallas as pl
from jax.experimental.pallas import tpu as pltpu
from jax.experimental.pallas import tpu_sc as plsc

N = 10000
E = 160000
D = 256
NG = 64

NC = 2
NS = 16
L = 16

CH = 32
NCHUNK_BASE = 312
N_PAD = 10016
ROWS_PER_SUB = N_PAD // NS

_MB = 5
BM = N // _MB
_EB = 80
BE = E // _EB


def _split128(z):
    return jnp.stack([z[:, :128], z[:, 128:]], axis=1).reshape(-1, 128)


def _hn_body(h_ref, w_ref, b_ref, ad_ref, hn_ref, hn2_ref, s_ref):
    hn = jnp.dot(h_ref[...], w_ref[...], preferred_element_type=jnp.float32)
    hn = hn + b_ref[...]
    hn_ref[...] = hn
    hn2_ref[...] = _split128(hn)
    s = jnp.sum(hn * ad_ref[0:1, :], axis=-1, keepdims=True)
    s_ref[...] = jnp.broadcast_to(s, s_ref.shape)


def _hn_call(h, W, b, ad2):
    return pl.pallas_call(
        _hn_body,
        grid=(_MB,),
        in_specs=[
            pl.BlockSpec((BM, D), lambda m: (m, 0)),
            pl.BlockSpec((D, D), lambda m: (0, 0)),
            pl.BlockSpec((1, D), lambda m: (0, 0)),
            pl.BlockSpec((1, D), lambda m: (0, 0)),
        ],
        out_specs=[
            pl.BlockSpec((BM, D), lambda m: (m, 0)),
            pl.BlockSpec((2 * BM, 128), lambda m: (m, 0)),
            pl.BlockSpec((BM, 128), lambda m: (m, 0)),
        ],
        out_shape=[
            jax.ShapeDtypeStruct((N, D), jnp.float32),
            jax.ShapeDtypeStruct((2 * N, 128), jnp.float32),
            jax.ShapeDtypeStruct((N, 128), jnp.float32),
        ],
    )(h, W, b, ad2)


def _e_body(ea_ref, w_ref, b_ref, e_ref):
    e = jnp.dot(ea_ref[...], w_ref[...], preferred_element_type=jnp.float32)
    e_ref[...] = _split128(e + b_ref[...])


def _e_call(edge_attr, W, b):
    de = edge_attr.shape[1]
    return pl.pallas_call(
        _e_body,
        grid=(_EB,),
        in_specs=[
            pl.BlockSpec((BE, de), lambda m: (m, 0)),
            pl.BlockSpec((de, D), lambda m: (0, 0)),
            pl.BlockSpec((1, D), lambda m: (0, 0)),
        ],
        out_specs=pl.BlockSpec((2 * BE, 128), lambda m: (m, 0)),
        out_shape=jax.ShapeDtypeStruct((2 * E, 128), jnp.float32),
    )(edge_attr, W, b)


def _sc_edge_body(hn, e, src, dst, sdst, asrc, num, den,
                  rows_v0, rows_v1, e_v0, e_v1, out_v, den_v,
                  src_v0, src_v1, src2_v0, src2_v1, dst_v0, dst_v1,
                  sd_v0, sd_v1,
                  asrc_v, dots_v, w_v, acc, dacc,
                  sem_i0, sem_i1, sem_r0, sem_r1, sem_e0, sem_e1,
                  sem_s0, sem_s1):
    c = lax.axis_index("c")
    s = lax.axis_index("s")

    rows_b = (rows_v0, rows_v1)
    e_b = (e_v0, e_v1)
    src_b = (src_v0, src_v1)
    src2_b = (src2_v0, src2_v1)
    dst_b = (dst_v0, dst_v1)
    sd_b = (sd_v0, sd_v1)
    sem_i = (sem_i0, sem_i1)
    sem_r = (sem_r0, sem_r1)
    sem_e = (sem_e0, sem_e1)
    sem_s = (sem_s0, sem_s1)

    pltpu.sync_copy(asrc, asrc_v)

    zero = jnp.zeros((L,), jnp.float32)

    def _zrow(i, carry):
        for j in range(128 // L):
            out_v[i, pl.ds(j * L, L)] = zero
        den_v[i, :] = zero
        return carry

    lax.fori_loop(0, CH, _zrow, 0)
    nfull = ROWS_PER_SUB // CH
    for kk in range(nfull):
        r0 = s * ROWS_PER_SUB + kk * CH
        pltpu.sync_copy(out_v, acc.at[pl.ds(r0, CH)])
        pltpu.sync_copy(den_v, dacc.at[pl.ds(r0, CH)])
    rem = ROWS_PER_SUB - nfull * CH
    if rem:
        r0 = s * ROWS_PER_SUB + nfull * CH
        pltpu.sync_copy(out_v.at[pl.ds(0, rem)], acc.at[pl.ds(r0, rem)])
        pltpu.sync_copy(den_v.at[pl.ds(0, rem)], dacc.at[pl.ds(r0, rem)])
    plsc.subcore_barrier()

    asrc_vals = [asrc_v[pl.ds(L * j, L)] for j in range(D // L)]
    iota_l = lax.iota(jnp.int32, L) * L
    halfoff = c
    nchunk = NCHUNK_BASE + jnp.where(s < 8, 1, 0)
    base0 = (s * NCHUNK_BASE + jnp.minimum(s, 8)) * CH

    def idx_copy_start(k, b):
        base = base0 + k * CH
        pltpu.async_copy(src.at[pl.ds(base, CH)], src_b[b], sem_i[b])
        pltpu.async_copy(dst.at[pl.ds(base, CH)], dst_b[b], sem_i[b])

    def idx_copy_wait(k, b):
        base = base0 + k * CH
        pltpu.make_async_copy(src.at[pl.ds(base, CH)], src_b[b], sem_i[b]).wait()
        pltpu.make_async_copy(dst.at[pl.ds(base, CH)], dst_b[b], sem_i[b]).wait()

    iota2 = lax.iota(jnp.int32, L) * 2

    def gather_start(k, b):
        base = base0 + k * CH
        for g in range(CH // L):
            sv = src_b[b][pl.ds(g * L, L)] * 2
            plsc.store_scatter(src2_b[b], [iota2 + (2 * L * g)], sv)
            plsc.store_scatter(src2_b[b], [iota2 + (2 * L * g + 1)], sv + 1)
        pltpu.async_copy(hn.at[src2_b[b]], rows_b[b], sem_r[b])
        pltpu.async_copy(e.at[pl.ds(2 * base, 2 * CH)], e_b[b], sem_e[b])
        pltpu.async_copy(sdst.at[dst_b[b]], sd_b[b], sem_s[b])

    def gather_wait(k, b):
        base = base0 + k * CH
        pltpu.make_async_copy(hn.at[src2_b[b]], rows_b[b], sem_r[b]).wait()
        pltpu.make_async_copy(e.at[pl.ds(2 * base, 2 * CH)], e_b[b], sem_e[b]).wait()
        pltpu.make_async_copy(sdst.at[dst_b[b]], sd_b[b], sem_s[b]).wait()

    def compute_chunk(b):
        rows_v = rows_b[b]
        e_v = e_b[b]
        sd_v = sd_b[b]

        def group_body(g, gc):
            eb = g * L
            for i in range(L):
                acc_i = zero
                for j in range(D // L):
                    r = 2 * (eb + i) + j // 8
                    hv = rows_v[r, pl.ds(L * (j % 8), L)]
                    ev = e_v[r, pl.ds(L * (j % 8), L)]
                    m = hv + ev
                    m = jnp.maximum(m, 0.2 * m)
                    acc_i = acc_i + m * asrc_vals[j]
                dots_v[pl.ds(i * L, L)] = acc_i
            tot = zero
            for cc in range(L):
                tot = tot + plsc.load_gather(dots_v, [iota_l + cc])
            sd = sd_v[pl.ds(eb, L)]
            logit = tot + sd
            logit = jnp.maximum(logit, 0.2 * logit)
            w_v[pl.ds(eb, L)] = jnp.exp(logit)
            return gc

        lax.fori_loop(0, CH // L, group_body, 0)

        def edge_body(i, ec):
            wi = plsc.load_gather(w_v, [jnp.zeros((L,), jnp.int32) + i])
            r = 2 * i + halfoff
            for j in range(128 // L):
                hv = rows_v[r, pl.ds(L * j, L)]
                ev = e_v[r, pl.ds(L * j, L)]
                m = hv + ev
                m = jnp.maximum(m, 0.2 * m)
                out_v[i, pl.ds(L * j, L)] = m * wi
            den_v[i, :] = wi
            return ec

        lax.fori_loop(0, CH, edge_body, 0)

    idx_copy_start(0, 0)
    idx_copy_wait(0, 0)
    gather_start(0, 0)
    idx_copy_start(1, 1)

    def chunk_pair(t, carry):
        for b in range(2):
            k = 2 * t + b
            nb = 1 - b

            @pl.when(k + 1 < nchunk)
            def _():
                idx_copy_wait(k + 1, nb)
                gather_start(k + 1, nb)

            @pl.when(k < nchunk)
            def _():
                gather_wait(k, b)
                compute_chunk(b)
                pltpu.sync_copy(out_v, acc.at[dst_b[b]], add=True)
                pltpu.sync_copy(den_v, dacc.at[dst_b[b]], add=True)

            @pl.when(k + 2 < nchunk)
            def _():
                idx_copy_start(k + 2, b)
        return carry

    lax.fori_loop(0, (NCHUNK_BASE + 2) // 2, chunk_pair, 0)
    plsc.subcore_barrier()

    r0 = s * ROWS_PER_SUB
    pltpu.sync_copy(acc.at[pl.ds(r0, ROWS_PER_SUB)],
                    num.at[c, pl.ds(r0, ROWS_PER_SUB)])
    pltpu.sync_copy(dacc.at[pl.ds(r0, ROWS_PER_SUB)],
                    den.at[c, pl.ds(r0, ROWS_PER_SUB)])


def _sc_edge_call(hn_b, e_b, src, dst, sdst, asrc):
    mesh = plsc.VectorSubcoreMesh(core_axis_name="c", subcore_axis_name="s")
    f = pl.kernel(
        _sc_edge_body,
        mesh=mesh,
        compiler_params=pltpu.CompilerParams(
            use_tc_tiling_on_sc=False, needs_layout_passes=False),
        out_type=[
            jax.ShapeDtypeStruct((NC, N_PAD, 128), jnp.float32),
            jax.ShapeDtypeStruct((NC, N_PAD, L), jnp.float32),
        ],
        scratch_types=[
            pltpu.VMEM((2 * CH, 128), jnp.float32),
            pltpu.VMEM((2 * CH, 128), jnp.float32),
            pltpu.VMEM((2 * CH, 128), jnp.float32),
            pltpu.VMEM((2 * CH, 128), jnp.float32),
            pltpu.VMEM((CH, 128), jnp.float32),
            pltpu.VMEM((CH, L), jnp.float32),
            pltpu.VMEM((CH,), jnp.int32),
            pltpu.VMEM((CH,), jnp.int32),
            pltpu.VMEM((2 * CH,), jnp.int32),
            pltpu.VMEM((2 * CH,), jnp.int32),
            pltpu.VMEM((CH,), jnp.int32),
            pltpu.VMEM((CH,), jnp.int32),
            pltpu.VMEM((CH,), jnp.float32),
            pltpu.VMEM((CH,), jnp.float32),
            pltpu.VMEM((D,), jnp.float32),
            pltpu.VMEM((L * L,), jnp.float32),
            pltpu.VMEM((CH,), jnp.float32),
            pltpu.VMEM_SHARED((N_PAD, 128), jnp.float32),
            pltpu.VMEM_SHARED((N_PAD, L), jnp.float32),
            pltpu.SemaphoreType.DMA,
            pltpu.SemaphoreType.DMA,
            pltpu.SemaphoreType.DMA,
            pltpu.SemaphoreType.DMA,
            pltpu.SemaphoreType.DMA,
            pltpu.SemaphoreType.DMA,
            pltpu.SemaphoreType.DMA,
            pltpu.SemaphoreType.DMA,
        ],
    )
    return f(hn_b, e_b, src, dst, sdst, asrc)


def _gru_math(x, h, wih_ref, whh_ref, bih_ref, bhh_ref):
    gi = jnp.dot(x, wih_ref[...], preferred_element_type=jnp.float32) + bih_ref[...]
    gh = jnp.dot(h, whh_ref[...], preferred_element_type=jnp.float32) + bhh_ref[...]
    dh = h.shape[-1]
    r = jax.nn.sigmoid(gi[:, :dh] + gh[:, :dh])
    z = jax.nn.sigmoid(gi[:, dh:2 * dh] + gh[:, dh:2 * dh])
    n = jnp.tanh(gi[:, 2 * dh:] + r * gh[:, 2 * dh:])
    return (1.0 - z) * n + z * h


def _gru_body(a0_ref, a1_ref, dn_ref, hn_ref, wih_ref, whh_ref, bih_ref,
              bhh_ref, h_ref):
    den = dn_ref[0][:, 0:1] + 1e-16
    num = jnp.concatenate([a0_ref[0], a1_ref[0]], axis=1)
    agg = num / den
    hn = hn_ref[...]
    h_ref[...] = jax.nn.relu(_gru_math(agg, hn, wih_ref, whh_ref, bih_ref, bhh_ref))


def _gru_call(num, den, hn, gp):
    return pl.pallas_call(
        _gru_body,
        grid=(_MB,),
        in_specs=[
            pl.BlockSpec((1, BM, 128), lambda m: (0, m, 0)),
            pl.BlockSpec((1, BM, 128), lambda m: (1, m, 0)),
            pl.BlockSpec((1, BM, L), lambda m: (0, m, 0)),
            pl.BlockSpec((BM, D), lambda m: (m, 0)),
            pl.BlockSpec((D, 3 * D), lambda m: (0, 0)),
            pl.BlockSpec((D, 3 * D), lambda m: (0, 0)),
            pl.BlockSpec((1, 3 * D), lambda m: (0, 0)),
            pl.BlockSpec((1, 3 * D), lambda m: (0, 0)),
        ],
        out_specs=pl.BlockSpec((BM, D), lambda m: (m, 0)),
        out_shape=jax.ShapeDtypeStruct((N, D), jnp.float32),
    )(num, num, den, hn, gp["W_ih"], gp["W_hh"],
      gp["b_ih"].reshape(1, -1), gp["b_hh"].reshape(1, -1))


def _head_body(h_ref, b_ref, wih, whh, bih, bhh,
               W1, b1, g1, be1, W2, b2, g2, be2, We, bee, Wo, bo,
               o_ref, pool_ref):
    m = pl.program_id(0)

    @pl.when(m == 0)
    def _():
        pool_ref[...] = jnp.zeros_like(pool_ref)

    onehot = (lax.broadcasted_iota(jnp.int32, (NG, BM), 0) == b_ref[0]).astype(jnp.float32)
    pool_ref[...] += jnp.dot(onehot, h_ref[...], preferred_element_type=jnp.float32,
                             precision=lax.Precision.HIGHEST)

    @pl.when(m == _MB - 1)
    def _():
        out = jax.nn.relu(pool_ref[...])
        for _ in range(2):
            out = jax.nn.relu(_gru_math(out, out, wih, whh, bih, bhh))
        for W, b, g, be in ((W1, b1, g1, be1), (W2, b2, g2, be2)):
            z = jnp.dot(out, W[...], preferred_element_type=jnp.float32) + b[...]
            mu = jnp.mean(z, axis=-1, keepdims=True)
            var = jnp.mean((z - mu) ** 2, axis=-1, keepdims=True)
            z = (z - mu) / jnp.sqrt(var + 1e-5) * g[...] + be[...]
            out = jax.nn.relu(z)
        emb = jnp.dot(out, We[...], preferred_element_type=jnp.float32) + bee[...]
        o_ref[...] = jnp.dot(emb, Wo[...], preferred_element_type=jnp.float32) + bo[...]


def _head_call(h, batch3d, params):
    l0, l1 = params["lin"]
    gp = params["mol_gru"]
    nt = params["out"]["W"].shape[1]
    d1 = l0["lin"]["W"].shape[1]
    d2 = l1["lin"]["W"].shape[1]

    def full2d(r, c):
        return pl.BlockSpec((r, c), lambda m: (0, 0))

    return pl.pallas_call(
        _head_body,
        grid=(_MB,),
        in_specs=[
            pl.BlockSpec((BM, D), lambda m: (m, 0)),
            pl.BlockSpec((1, 1, BM), lambda m: (m, 0, 0)),
            full2d(D, 3 * D), full2d(D, 3 * D), full2d(1, 3 * D), full2d(1, 3 * D),
            full2d(D, d1), full2d(1, d1), full2d(1, d1), full2d(1, d1),
            full2d(d1, d2), full2d(1, d2), full2d(1, d2), full2d(1, d2),
            full2d(d2, d2), full2d(1, d2),
            full2d(d2, nt), full2d(1, nt),
        ],
        out_specs=pl.BlockSpec((NG, nt), lambda m: (0, 0)),
        out_shape=jax.ShapeDtypeStruct((NG, nt), jnp.float32),
        scratch_shapes=[pltpu.VMEM((NG, D), jnp.float32)],
    )(h, batch3d,
      gp["W_ih"], gp["W_hh"], gp["b_ih"].reshape(1, -1), gp["b_hh"].reshape(1, -1),
      l0["lin"]["W"], l0["lin"]["b"].reshape(1, -1),
      l0["gamma"].reshape(1, -1), l0["beta"].reshape(1, -1),
      l1["lin"]["W"], l1["lin"]["b"].reshape(1, -1),
      l1["gamma"].reshape(1, -1), l1["beta"].reshape(1, -1),
      params["emb"]["W"], params["emb"]["b"].reshape(1, -1),
      params["out"]["W"], params["out"]["b"].reshape(1, -1))


def kernel(x, edge_index, edge_attr, batch, params):
    src = edge_index[0].astype(jnp.int32)
    dst = edge_index[1].astype(jnp.int32)
    batch3d = batch.astype(jnp.int32).reshape(_MB, 1, BM)

    h = x
    for lp in params["agg"]:
        ad2 = lp["a_dst"].reshape(1, -1)
        hn, hn2, s2d = _hn_call(h, lp["node"]["W"], lp["node"]["b"].reshape(1, -1), ad2)
        sdst = s2d[:, 0]
        e = _e_call(edge_attr, lp["edge"]["W"], lp["edge"]["b"].reshape(1, -1))
        num, den = _sc_edge_call(hn2, e, src, dst, sdst, lp["a_src"])
        h = _gru_call(num, den, hn, lp["gru"])

    return _head_call(h, batch3d, params)


# speedup vs baseline: 3.4855x; 1.3497x over previous
import functools

import jax
import jax.numpy as jnp
from jax import lax
from jax.experimental import pallas as pl
from jax.experimental.pallas import tpu as pltpu
from jax.experimental.pallas import tpu_sc as plsc

N = 10000
E = 160000
D = 256
NG = 64

NC = 2
NS = 16
L = 16

CH = 32
NCHUNK_BASE = 312
N_PAD = 10016
ROWS_PER_SUB = N_PAD // NS

_MB = 5
BM = N // _MB
_EB = 80
BE = E // _EB


def _split128(z):
    return jnp.stack([z[:, :128], z[:, 128:]], axis=1).reshape(-1, 128)


def _hn_body(h_ref, w_ref, b_ref, ad_ref, hn_ref, hn2_ref, s_ref):
    hn = jnp.dot(h_ref[...], w_ref[...], preferred_element_type=jnp.float32)
    hn = hn + b_ref[...]
    hn_ref[...] = hn
    hn2_ref[...] = _split128(hn)
    s = jnp.sum(hn * ad_ref[0:1, :], axis=-1, keepdims=True)
    s_ref[...] = jnp.broadcast_to(s, s_ref.shape)


def _hn_call(h, W, b, ad2):
    return pl.pallas_call(
        _hn_body,
        grid=(_MB,),
        in_specs=[
            pl.BlockSpec((BM, D), lambda m: (m, 0)),
            pl.BlockSpec((D, D), lambda m: (0, 0)),
            pl.BlockSpec((1, D), lambda m: (0, 0)),
            pl.BlockSpec((1, D), lambda m: (0, 0)),
        ],
        out_specs=[
            pl.BlockSpec((BM, D), lambda m: (m, 0)),
            pl.BlockSpec((2 * BM, 128), lambda m: (m, 0)),
            pl.BlockSpec((BM, 128), lambda m: (m, 0)),
        ],
        out_shape=[
            jax.ShapeDtypeStruct((N, D), jnp.float32),
            jax.ShapeDtypeStruct((2 * N, 128), jnp.float32),
            jax.ShapeDtypeStruct((N, 128), jnp.float32),
        ],
    )(h, W, b, ad2)


def _e_body(ea_ref, w_ref, b_ref, e_ref):
    e = jnp.dot(ea_ref[...], w_ref[...], preferred_element_type=jnp.float32)
    e_ref[...] = _split128(e + b_ref[...])


def _e_call(edge_attr, W, b):
    de = edge_attr.shape[1]
    return pl.pallas_call(
        _e_body,
        grid=(_EB,),
        in_specs=[
            pl.BlockSpec((BE, de), lambda m: (m, 0)),
            pl.BlockSpec((de, D), lambda m: (0, 0)),
            pl.BlockSpec((1, D), lambda m: (0, 0)),
        ],
        out_specs=pl.BlockSpec((2 * BE, 128), lambda m: (m, 0)),
        out_shape=jax.ShapeDtypeStruct((2 * E, 128), jnp.float32),
    )(edge_attr, W, b)


def _sc_edge_body(hn, e, src, dst, sdst, asrc, num, den,
                  rows_v0, rows_v1, e_v0, e_v1, out_v, den_v,
                  src_v0, src_v1, src2_v0, src2_v1, dst_v0, dst_v1,
                  sd_v0, sd_v1,
                  asrc_v, dots_v, w_v, acc, dacc,
                  sem_i0, sem_i1, sem_r0, sem_r1, sem_e0, sem_e1,
                  sem_s0, sem_s1):
    c = lax.axis_index("c")
    s = lax.axis_index("s")

    rows_b = (rows_v0, rows_v1)
    e_b = (e_v0, e_v1)
    src_b = (src_v0, src_v1)
    src2_b = (src2_v0, src2_v1)
    dst_b = (dst_v0, dst_v1)
    sd_b = (sd_v0, sd_v1)
    sem_i = (sem_i0, sem_i1)
    sem_r = (sem_r0, sem_r1)
    sem_e = (sem_e0, sem_e1)
    sem_s = (sem_s0, sem_s1)

    pltpu.sync_copy(asrc, asrc_v)

    zero = jnp.zeros((L,), jnp.float32)

    def _zrow(i, carry):
        for j in range(128 // L):
            out_v[i, pl.ds(j * L, L)] = zero
        den_v[i, :] = zero
        return carry

    lax.fori_loop(0, CH, _zrow, 0)
    nfull = ROWS_PER_SUB // CH
    for kk in range(nfull):
        r0 = s * ROWS_PER_SUB + kk * CH
        pltpu.sync_copy(out_v, acc.at[pl.ds(r0, CH)])
        pltpu.sync_copy(den_v, dacc.at[pl.ds(r0, CH)])
    rem = ROWS_PER_SUB - nfull * CH
    if rem:
        r0 = s * ROWS_PER_SUB + nfull * CH
        pltpu.sync_copy(out_v.at[pl.ds(0, rem)], acc.at[pl.ds(r0, rem)])
        pltpu.sync_copy(den_v.at[pl.ds(0, rem)], dacc.at[pl.ds(r0, rem)])
    plsc.subcore_barrier()

    asrc_vals = [asrc_v[pl.ds(L * j, L)] for j in range(D // L)]
    iota_l = lax.iota(jnp.int32, L) * L
    halfoff = c
    nchunk = NCHUNK_BASE + jnp.where(s < 8, 1, 0)
    base0 = (s * NCHUNK_BASE + jnp.minimum(s, 8)) * CH

    def idx_copy_start(k, b):
        base = base0 + k * CH
        pltpu.async_copy(src.at[pl.ds(base, CH)], src_b[b], sem_i[b])
        pltpu.async_copy(dst.at[pl.ds(base, CH)], dst_b[b], sem_i[b])

    def idx_copy_wait(k, b):
        base = base0 + k * CH
        pltpu.make_async_copy(src.at[pl.ds(base, CH)], src_b[b], sem_i[b]).wait()
        pltpu.make_async_copy(dst.at[pl.ds(base, CH)], dst_b[b], sem_i[b]).wait()

    iota2 = lax.iota(jnp.int32, L) * 2

    def gather_start(k, b):
        base = base0 + k * CH
        for g in range(CH // L):
            sv = src_b[b][pl.ds(g * L, L)] * 2
            plsc.store_scatter(src2_b[b], [iota2 + (2 * L * g)], sv)
            plsc.store_scatter(src2_b[b], [iota2 + (2 * L * g + 1)], sv + 1)
        pltpu.async_copy(hn.at[src2_b[b]], rows_b[b], sem_r[b])
        pltpu.async_copy(e.at[pl.ds(2 * base, 2 * CH)], e_b[b], sem_e[b])
        pltpu.async_copy(sdst.at[dst_b[b]], sd_b[b], sem_s[b])

    def gather_wait(k, b):
        base = base0 + k * CH
        pltpu.make_async_copy(hn.at[src2_b[b]], rows_b[b], sem_r[b]).wait()
        pltpu.make_async_copy(e.at[pl.ds(2 * base, 2 * CH)], e_b[b], sem_e[b]).wait()
        pltpu.make_async_copy(sdst.at[dst_b[b]], sd_b[b], sem_s[b]).wait()

    def compute_chunk(b):
        rows_v = rows_b[b]
        e_v = e_b[b]
        sd_v = sd_b[b]

        @plsc.parallel_loop(0, CH // L, unroll=1)
        def _(g):
            eb = g * L
            db = g * (L * L)
            for i in range(L):
                acc_i = zero
                for j in range(D // L):
                    r = 2 * (eb + i) + j // 8
                    hv = rows_v[r, pl.ds(L * (j % 8), L)]
                    ev = e_v[r, pl.ds(L * (j % 8), L)]
                    m = hv + ev
                    m = jnp.maximum(m, 0.2 * m)
                    acc_i = acc_i + m * asrc_vals[j]
                dots_v[pl.ds(db + i * L, L)] = acc_i
            tot = zero
            for cc in range(L):
                tot = tot + plsc.load_gather(dots_v, [db + iota_l + cc])
            sd = sd_v[pl.ds(eb, L)]
            logit = tot + sd
            logit = jnp.maximum(logit, 0.2 * logit)
            w_v[pl.ds(eb, L)] = jnp.exp(logit)

        @plsc.parallel_loop(0, CH, unroll=2)
        def _(i):
            wi = plsc.load_gather(w_v, [jnp.zeros((L,), jnp.int32) + i])
            r = 2 * i + halfoff
            for j in range(128 // L):
                hv = rows_v[r, pl.ds(L * j, L)]
                ev = e_v[r, pl.ds(L * j, L)]
                m = hv + ev
                m = jnp.maximum(m, 0.2 * m)
                out_v[i, pl.ds(L * j, L)] = m * wi
            den_v[i, :] = wi

    idx_copy_start(0, 0)
    idx_copy_wait(0, 0)
    gather_start(0, 0)
    idx_copy_start(1, 1)

    def chunk_pair(t, carry):
        for b in range(2):
            k = 2 * t + b
            nb = 1 - b

            @pl.when(k + 1 < nchunk)
            def _():
                idx_copy_wait(k + 1, nb)
                gather_start(k + 1, nb)

            @pl.when(k < nchunk)
            def _():
                gather_wait(k, b)
                compute_chunk(b)
                pltpu.sync_copy(out_v, acc.at[dst_b[b]], add=True)
                pltpu.sync_copy(den_v, dacc.at[dst_b[b]], add=True)

            @pl.when(k + 2 < nchunk)
            def _():
                idx_copy_start(k + 2, b)
        return carry

    lax.fori_loop(0, (NCHUNK_BASE + 2) // 2, chunk_pair, 0)
    plsc.subcore_barrier()

    r0 = s * ROWS_PER_SUB
    pltpu.sync_copy(acc.at[pl.ds(r0, ROWS_PER_SUB)],
                    num.at[c, pl.ds(r0, ROWS_PER_SUB)])
    pltpu.sync_copy(dacc.at[pl.ds(r0, ROWS_PER_SUB)],
                    den.at[c, pl.ds(r0, ROWS_PER_SUB)])


def _sc_edge_call(hn_b, e_b, src, dst, sdst, asrc):
    mesh = plsc.VectorSubcoreMesh(core_axis_name="c", subcore_axis_name="s")
    f = pl.kernel(
        _sc_edge_body,
        mesh=mesh,
        compiler_params=pltpu.CompilerParams(
            use_tc_tiling_on_sc=False, needs_layout_passes=False),
        out_type=[
            jax.ShapeDtypeStruct((NC, N_PAD, 128), jnp.float32),
            jax.ShapeDtypeStruct((NC, N_PAD, L), jnp.float32),
        ],
        scratch_types=[
            pltpu.VMEM((2 * CH, 128), jnp.float32),
            pltpu.VMEM((2 * CH, 128), jnp.float32),
            pltpu.VMEM((2 * CH, 128), jnp.float32),
            pltpu.VMEM((2 * CH, 128), jnp.float32),
            pltpu.VMEM((CH, 128), jnp.float32),
            pltpu.VMEM((CH, L), jnp.float32),
            pltpu.VMEM((CH,), jnp.int32),
            pltpu.VMEM((CH,), jnp.int32),
            pltpu.VMEM((2 * CH,), jnp.int32),
            pltpu.VMEM((2 * CH,), jnp.int32),
            pltpu.VMEM((CH,), jnp.int32),
            pltpu.VMEM((CH,), jnp.int32),
            pltpu.VMEM((CH,), jnp.float32),
            pltpu.VMEM((CH,), jnp.float32),
            pltpu.VMEM((D,), jnp.float32),
            pltpu.VMEM((CH * L,), jnp.float32),
            pltpu.VMEM((CH,), jnp.float32),
            pltpu.VMEM_SHARED((N_PAD, 128), jnp.float32),
            pltpu.VMEM_SHARED((N_PAD, L), jnp.float32),
            pltpu.SemaphoreType.DMA,
            pltpu.SemaphoreType.DMA,
            pltpu.SemaphoreType.DMA,
            pltpu.SemaphoreType.DMA,
            pltpu.SemaphoreType.DMA,
            pltpu.SemaphoreType.DMA,
            pltpu.SemaphoreType.DMA,
            pltpu.SemaphoreType.DMA,
        ],
    )
    return f(hn_b, e_b, src, dst, sdst, asrc)


def _gru_math(x, h, wih_ref, whh_ref, bih_ref, bhh_ref):
    gi = jnp.dot(x, wih_ref[...], preferred_element_type=jnp.float32) + bih_ref[...]
    gh = jnp.dot(h, whh_ref[...], preferred_element_type=jnp.float32) + bhh_ref[...]
    dh = h.shape[-1]
    r = jax.nn.sigmoid(gi[:, :dh] + gh[:, :dh])
    z = jax.nn.sigmoid(gi[:, dh:2 * dh] + gh[:, dh:2 * dh])
    n = jnp.tanh(gi[:, 2 * dh:] + r * gh[:, 2 * dh:])
    return (1.0 - z) * n + z * h


def _gru_body(a0_ref, a1_ref, dn_ref, hn_ref, wih_ref, whh_ref, bih_ref,
              bhh_ref, h_ref):
    den = dn_ref[0][:, 0:1] + 1e-16
    num = jnp.concatenate([a0_ref[0], a1_ref[0]], axis=1)
    agg = num / den
    hn = hn_ref[...]
    h_ref[...] = jax.nn.relu(_gru_math(agg, hn, wih_ref, whh_ref, bih_ref, bhh_ref))


def _gru_call(num, den, hn, gp):
    return pl.pallas_call(
        _gru_body,
        grid=(_MB,),
        in_specs=[
            pl.BlockSpec((1, BM, 128), lambda m: (0, m, 0)),
            pl.BlockSpec((1, BM, 128), lambda m: (1, m, 0)),
            pl.BlockSpec((1, BM, L), lambda m: (0, m, 0)),
            pl.BlockSpec((BM, D), lambda m: (m, 0)),
            pl.BlockSpec((D, 3 * D), lambda m: (0, 0)),
            pl.BlockSpec((D, 3 * D), lambda m: (0, 0)),
            pl.BlockSpec((1, 3 * D), lambda m: (0, 0)),
            pl.BlockSpec((1, 3 * D), lambda m: (0, 0)),
        ],
        out_specs=pl.BlockSpec((BM, D), lambda m: (m, 0)),
        out_shape=jax.ShapeDtypeStruct((N, D), jnp.float32),
    )(num, num, den, hn, gp["W_ih"], gp["W_hh"],
      gp["b_ih"].reshape(1, -1), gp["b_hh"].reshape(1, -1))


def _head_body(h_ref, b_ref, wih, whh, bih, bhh,
               W1, b1, g1, be1, W2, b2, g2, be2, We, bee, Wo, bo,
               o_ref, pool_ref):
    m = pl.program_id(0)

    @pl.when(m == 0)
    def _():
        pool_ref[...] = jnp.zeros_like(pool_ref)

    onehot = (lax.broadcasted_iota(jnp.int32, (NG, BM), 0) == b_ref[0]).astype(jnp.float32)
    pool_ref[...] += jnp.dot(onehot, h_ref[...], preferred_element_type=jnp.float32,
                             precision=lax.Precision.HIGHEST)

    @pl.when(m == _MB - 1)
    def _():
        out = jax.nn.relu(pool_ref[...])
        for _ in range(2):
            out = jax.nn.relu(_gru_math(out, out, wih, whh, bih, bhh))
        for W, b, g, be in ((W1, b1, g1, be1), (W2, b2, g2, be2)):
            z = jnp.dot(out, W[...], preferred_element_type=jnp.float32) + b[...]
            mu = jnp.mean(z, axis=-1, keepdims=True)
            var = jnp.mean((z - mu) ** 2, axis=-1, keepdims=True)
            z = (z - mu) / jnp.sqrt(var + 1e-5) * g[...] + be[...]
            out = jax.nn.relu(z)
        emb = jnp.dot(out, We[...], preferred_element_type=jnp.float32) + bee[...]
        o_ref[...] = jnp.dot(emb, Wo[...], preferred_element_type=jnp.float32) + bo[...]


def _head_call(h, batch3d, params):
    l0, l1 = params["lin"]
    gp = params["mol_gru"]
    nt = params["out"]["W"].shape[1]
    d1 = l0["lin"]["W"].shape[1]
    d2 = l1["lin"]["W"].shape[1]

    def full2d(r, c):
        return pl.BlockSpec((r, c), lambda m: (0, 0))

    return pl.pallas_call(
        _head_body,
        grid=(_MB,),
        in_specs=[
            pl.BlockSpec((BM, D), lambda m: (m, 0)),
            pl.BlockSpec((1, 1, BM), lambda m: (m, 0, 0)),
            full2d(D, 3 * D), full2d(D, 3 * D), full2d(1, 3 * D), full2d(1, 3 * D),
            full2d(D, d1), full2d(1, d1), full2d(1, d1), full2d(1, d1),
            full2d(d1, d2), full2d(1, d2), full2d(1, d2), full2d(1, d2),
            full2d(d2, d2), full2d(1, d2),
            full2d(d2, nt), full2d(1, nt),
        ],
        out_specs=pl.BlockSpec((NG, nt), lambda m: (0, 0)),
        out_shape=jax.ShapeDtypeStruct((NG, nt), jnp.float32),
        scratch_shapes=[pltpu.VMEM((NG, D), jnp.float32)],
    )(h, batch3d,
      gp["W_ih"], gp["W_hh"], gp["b_ih"].reshape(1, -1), gp["b_hh"].reshape(1, -1),
      l0["lin"]["W"], l0["lin"]["b"].reshape(1, -1),
      l0["gamma"].reshape(1, -1), l0["beta"].reshape(1, -1),
      l1["lin"]["W"], l1["lin"]["b"].reshape(1, -1),
      l1["gamma"].reshape(1, -1), l1["beta"].reshape(1, -1),
      params["emb"]["W"], params["emb"]["b"].reshape(1, -1),
      params["out"]["W"], params["out"]["b"].reshape(1, -1))


def kernel(x, edge_index, edge_attr, batch, params):
    src = edge_index[0].astype(jnp.int32)
    dst = edge_index[1].astype(jnp.int32)
    batch3d = batch.astype(jnp.int32).reshape(_MB, 1, BM)

    h = x
    for lp in params["agg"]:
        ad2 = lp["a_dst"].reshape(1, -1)
        hn, hn2, s2d = _hn_call(h, lp["node"]["W"], lp["node"]["b"].reshape(1, -1), ad2)
        sdst = s2d[:, 0]
        e = _e_call(edge_attr, lp["edge"]["W"], lp["edge"]["b"].reshape(1, -1))
        num, den = _sc_edge_call(hn2, e, src, dst, sdst, lp["a_src"])
        h = _gru_call(num, den, hn, lp["gru"])

    return _head_call(h, batch3d, params)


# speedup vs baseline: 3.5088x; 1.0067x over previous
import functools

import jax
import jax.numpy as jnp
from jax import lax
from jax.experimental import pallas as pl
from jax.experimental.pallas import tpu as pltpu
from jax.experimental.pallas import tpu_sc as plsc

N = 10000
E = 160000
D = 256
NG = 64

NC = 2
NS = 16
L = 16

CH = 32
NCHUNK_BASE = 312
N_PAD = 10016
ROWS_PER_SUB = N_PAD // NS

_MB = 5
BM = N // _MB
_EB = 80
BE = E // _EB


def _split128(z):
    return jnp.stack([z[:, :128], z[:, 128:]], axis=1).reshape(-1, 128)


def _hn_body(h_ref, w_ref, b_ref, ad_ref, hn_ref, hn2_ref, s_ref):
    hn = jnp.dot(h_ref[...], w_ref[...], preferred_element_type=jnp.float32)
    hn = hn + b_ref[...]
    hn_ref[...] = hn
    hn2_ref[...] = _split128(hn)
    s = jnp.sum(hn * ad_ref[0:1, :], axis=-1, keepdims=True)
    s_ref[...] = jnp.broadcast_to(s, s_ref.shape)


def _hn_call(h, W, b, ad2):
    return pl.pallas_call(
        _hn_body,
        grid=(_MB,),
        in_specs=[
            pl.BlockSpec((BM, D), lambda m: (m, 0)),
            pl.BlockSpec((D, D), lambda m: (0, 0)),
            pl.BlockSpec((1, D), lambda m: (0, 0)),
            pl.BlockSpec((1, D), lambda m: (0, 0)),
        ],
        out_specs=[
            pl.BlockSpec((BM, D), lambda m: (m, 0)),
            pl.BlockSpec((2 * BM, 128), lambda m: (m, 0)),
            pl.BlockSpec((BM, 128), lambda m: (m, 0)),
        ],
        out_shape=[
            jax.ShapeDtypeStruct((N, D), jnp.float32),
            jax.ShapeDtypeStruct((2 * N, 128), jnp.float32),
            jax.ShapeDtypeStruct((N, 128), jnp.float32),
        ],
    )(h, W, b, ad2)


def _e_body(ea_ref, w_ref, b_ref, e_ref):
    e = jnp.dot(ea_ref[...], w_ref[...], preferred_element_type=jnp.float32)
    e_ref[...] = _split128(e + b_ref[...])


def _e_call(edge_attr, W, b):
    de = edge_attr.shape[1]
    return pl.pallas_call(
        _e_body,
        grid=(_EB,),
        in_specs=[
            pl.BlockSpec((BE, de), lambda m: (m, 0)),
            pl.BlockSpec((de, D), lambda m: (0, 0)),
            pl.BlockSpec((1, D), lambda m: (0, 0)),
        ],
        out_specs=pl.BlockSpec((2 * BE, 128), lambda m: (m, 0)),
        out_shape=jax.ShapeDtypeStruct((2 * E, 128), jnp.float32),
    )(edge_attr, W, b)


def _sc_edge_body(hn, e, src, dst, sdst, asrc, num, den,
                  rows_v0, rows_v1, e_v0, e_v1, out_v, den_v,
                  src_v0, src_v1, src2_v0, src2_v1, dst_v0, dst_v1,
                  sd_v0, sd_v1,
                  asrc_v, dots_v, w_v, acc, dacc,
                  sem_i0, sem_i1, sem_r0, sem_r1, sem_e0, sem_e1,
                  sem_s0, sem_s1):
    c = lax.axis_index("c")
    s = lax.axis_index("s")

    rows_b = (rows_v0, rows_v1)
    e_b = (e_v0, e_v1)
    src_b = (src_v0, src_v1)
    src2_b = (src2_v0, src2_v1)
    dst_b = (dst_v0, dst_v1)
    sd_b = (sd_v0, sd_v1)
    sem_i = (sem_i0, sem_i1)
    sem_r = (sem_r0, sem_r1)
    sem_e = (sem_e0, sem_e1)
    sem_s = (sem_s0, sem_s1)

    pltpu.sync_copy(asrc, asrc_v)

    zero = jnp.zeros((L,), jnp.float32)

    def _zrow(i, carry):
        for j in range(128 // L):
            out_v[i, pl.ds(j * L, L)] = zero
        den_v[i, :] = zero
        return carry

    lax.fori_loop(0, CH, _zrow, 0)
    nfull = ROWS_PER_SUB // CH
    for kk in range(nfull):
        r0 = s * ROWS_PER_SUB + kk * CH
        pltpu.sync_copy(out_v, acc.at[pl.ds(r0, CH)])
        pltpu.sync_copy(den_v, dacc.at[pl.ds(r0, CH)])
    rem = ROWS_PER_SUB - nfull * CH
    if rem:
        r0 = s * ROWS_PER_SUB + nfull * CH
        pltpu.sync_copy(out_v.at[pl.ds(0, rem)], acc.at[pl.ds(r0, rem)])
        pltpu.sync_copy(den_v.at[pl.ds(0, rem)], dacc.at[pl.ds(r0, rem)])
    plsc.subcore_barrier()

    asrc_vals = [asrc_v[pl.ds(L * j, L)] for j in range(D // L)]
    iota_l = lax.iota(jnp.int32, L) * L
    halfoff = c
    nchunk = NCHUNK_BASE + jnp.where(s < 8, 1, 0)
    base0 = (s * NCHUNK_BASE + jnp.minimum(s, 8)) * CH

    def idx_copy_start(k, b):
        base = base0 + k * CH
        pltpu.async_copy(src.at[pl.ds(base, CH)], src_b[b], sem_i[b])
        pltpu.async_copy(dst.at[pl.ds(base, CH)], dst_b[b], sem_i[b])

    def idx_copy_wait(k, b):
        base = base0 + k * CH
        pltpu.make_async_copy(src.at[pl.ds(base, CH)], src_b[b], sem_i[b]).wait()
        pltpu.make_async_copy(dst.at[pl.ds(base, CH)], dst_b[b], sem_i[b]).wait()

    iota2 = lax.iota(jnp.int32, L) * 2

    def gather_start(k, b):
        base = base0 + k * CH
        for g in range(CH // L):
            sv = src_b[b][pl.ds(g * L, L)] * 2
            plsc.store_scatter(src2_b[b], [iota2 + (2 * L * g)], sv)
            plsc.store_scatter(src2_b[b], [iota2 + (2 * L * g + 1)], sv + 1)
        pltpu.async_copy(hn.at[src2_b[b]], rows_b[b], sem_r[b])
        pltpu.async_copy(e.at[pl.ds(2 * base, 2 * CH)], e_b[b], sem_e[b])
        pltpu.async_copy(sdst.at[dst_b[b]], sd_b[b], sem_s[b])

    def gather_wait(k, b):
        base = base0 + k * CH
        pltpu.make_async_copy(hn.at[src2_b[b]], rows_b[b], sem_r[b]).wait()
        pltpu.make_async_copy(e.at[pl.ds(2 * base, 2 * CH)], e_b[b], sem_e[b]).wait()
        pltpu.make_async_copy(sdst.at[dst_b[b]], sd_b[b], sem_s[b]).wait()

    def compute_chunk(b):
        rows_v = rows_b[b]
        e_v = e_b[b]
        sd_v = sd_b[b]

        @plsc.parallel_loop(0, CH // L, unroll=1)
        def _(g):
            eb = g * L
            db = g * (L * L)
            for i in range(L):
                acc_i = zero
                for j in range(D // L):
                    r = 2 * (eb + i) + j // 8
                    hv = rows_v[r, pl.ds(L * (j % 8), L)]
                    ev = e_v[r, pl.ds(L * (j % 8), L)]
                    m = hv + ev
                    m = jnp.maximum(m, 0.2 * m)
                    acc_i = acc_i + m * asrc_vals[j]
                dots_v[pl.ds(db + i * L, L)] = acc_i
            tot = zero
            for cc in range(L):
                tot = tot + plsc.load_gather(dots_v, [db + iota_l + cc])
            sd = sd_v[pl.ds(eb, L)]
            logit = tot + sd
            logit = jnp.maximum(logit, 0.2 * logit)
            w_v[pl.ds(eb, L)] = jnp.exp(logit)

        @plsc.parallel_loop(0, CH, unroll=4)
        def _(i):
            wi = plsc.load_gather(w_v, [jnp.zeros((L,), jnp.int32) + i])
            r = 2 * i + halfoff
            for j in range(128 // L):
                hv = rows_v[r, pl.ds(L * j, L)]
                ev = e_v[r, pl.ds(L * j, L)]
                m = hv + ev
                m = jnp.maximum(m, 0.2 * m)
                out_v[i, pl.ds(L * j, L)] = m * wi
            den_v[i, :] = wi

    idx_copy_start(0, 0)
    idx_copy_wait(0, 0)
    gather_start(0, 0)
    idx_copy_start(1, 1)

    def chunk_pair(t, carry):
        for b in range(2):
            k = 2 * t + b
            nb = 1 - b

            @pl.when(k + 1 < nchunk)
            def _():
                idx_copy_wait(k + 1, nb)
                gather_start(k + 1, nb)

            @pl.when(k < nchunk)
            def _():
                gather_wait(k, b)
                compute_chunk(b)
                pltpu.sync_copy(out_v, acc.at[dst_b[b]], add=True)
                pltpu.sync_copy(den_v, dacc.at[dst_b[b]], add=True)

            @pl.when(k + 2 < nchunk)
            def _():
                idx_copy_start(k + 2, b)
        return carry

    lax.fori_loop(0, (NCHUNK_BASE + 2) // 2, chunk_pair, 0)
    plsc.subcore_barrier()

    r0 = s * ROWS_PER_SUB
    pltpu.sync_copy(acc.at[pl.ds(r0, ROWS_PER_SUB)],
                    num.at[c, pl.ds(r0, ROWS_PER_SUB)])
    pltpu.sync_copy(dacc.at[pl.ds(r0, ROWS_PER_SUB)],
                    den.at[c, pl.ds(r0, ROWS_PER_SUB)])


def _sc_edge_call(hn_b, e_b, src, dst, sdst, asrc):
    mesh = plsc.VectorSubcoreMesh(core_axis_name="c", subcore_axis_name="s")
    f = pl.kernel(
        _sc_edge_body,
        mesh=mesh,
        compiler_params=pltpu.CompilerParams(
            use_tc_tiling_on_sc=False, needs_layout_passes=False),
        out_type=[
            jax.ShapeDtypeStruct((NC, N_PAD, 128), jnp.float32),
            jax.ShapeDtypeStruct((NC, N_PAD, L), jnp.float32),
        ],
        scratch_types=[
            pltpu.VMEM((2 * CH, 128), jnp.float32),
            pltpu.VMEM((2 * CH, 128), jnp.float32),
            pltpu.VMEM((2 * CH, 128), jnp.float32),
            pltpu.VMEM((2 * CH, 128), jnp.float32),
            pltpu.VMEM((CH, 128), jnp.float32),
            pltpu.VMEM((CH, L), jnp.float32),
            pltpu.VMEM((CH,), jnp.int32),
            pltpu.VMEM((CH,), jnp.int32),
            pltpu.VMEM((2 * CH,), jnp.int32),
            pltpu.VMEM((2 * CH,), jnp.int32),
            pltpu.VMEM((CH,), jnp.int32),
            pltpu.VMEM((CH,), jnp.int32),
            pltpu.VMEM((CH,), jnp.float32),
            pltpu.VMEM((CH,), jnp.float32),
            pltpu.VMEM((D,), jnp.float32),
            pltpu.VMEM((CH * L,), jnp.float32),
            pltpu.VMEM((CH,), jnp.float32),
            pltpu.VMEM_SHARED((N_PAD, 128), jnp.float32),
            pltpu.VMEM_SHARED((N_PAD, L), jnp.float32),
            pltpu.SemaphoreType.DMA,
            pltpu.SemaphoreType.DMA,
            pltpu.SemaphoreType.DMA,
            pltpu.SemaphoreType.DMA,
            pltpu.SemaphoreType.DMA,
            pltpu.SemaphoreType.DMA,
            pltpu.SemaphoreType.DMA,
            pltpu.SemaphoreType.DMA,
        ],
    )
    return f(hn_b, e_b, src, dst, sdst, asrc)


def _gru_math(x, h, wih_ref, whh_ref, bih_ref, bhh_ref):
    gi = jnp.dot(x, wih_ref[...], preferred_element_type=jnp.float32) + bih_ref[...]
    gh = jnp.dot(h, whh_ref[...], preferred_element_type=jnp.float32) + bhh_ref[...]
    dh = h.shape[-1]
    r = jax.nn.sigmoid(gi[:, :dh] + gh[:, :dh])
    z = jax.nn.sigmoid(gi[:, dh:2 * dh] + gh[:, dh:2 * dh])
    n = jnp.tanh(gi[:, 2 * dh:] + r * gh[:, 2 * dh:])
    return (1.0 - z) * n + z * h


def _gru_body(a0_ref, a1_ref, dn_ref, hn_ref, wih_ref, whh_ref, bih_ref,
              bhh_ref, h_ref):
    den = dn_ref[0][:, 0:1] + 1e-16
    num = jnp.concatenate([a0_ref[0], a1_ref[0]], axis=1)
    agg = num / den
    hn = hn_ref[...]
    h_ref[...] = jax.nn.relu(_gru_math(agg, hn, wih_ref, whh_ref, bih_ref, bhh_ref))


def _gru_call(num, den, hn, gp):
    return pl.pallas_call(
        _gru_body,
        grid=(_MB,),
        in_specs=[
            pl.BlockSpec((1, BM, 128), lambda m: (0, m, 0)),
            pl.BlockSpec((1, BM, 128), lambda m: (1, m, 0)),
            pl.BlockSpec((1, BM, L), lambda m: (0, m, 0)),
            pl.BlockSpec((BM, D), lambda m: (m, 0)),
            pl.BlockSpec((D, 3 * D), lambda m: (0, 0)),
            pl.BlockSpec((D, 3 * D), lambda m: (0, 0)),
            pl.BlockSpec((1, 3 * D), lambda m: (0, 0)),
            pl.BlockSpec((1, 3 * D), lambda m: (0, 0)),
        ],
        out_specs=pl.BlockSpec((BM, D), lambda m: (m, 0)),
        out_shape=jax.ShapeDtypeStruct((N, D), jnp.float32),
    )(num, num, den, hn, gp["W_ih"], gp["W_hh"],
      gp["b_ih"].reshape(1, -1), gp["b_hh"].reshape(1, -1))


def _head_body(h_ref, b_ref, wih, whh, bih, bhh,
               W1, b1, g1, be1, W2, b2, g2, be2, We, bee, Wo, bo,
               o_ref, pool_ref):
    m = pl.program_id(0)

    @pl.when(m == 0)
    def _():
        pool_ref[...] = jnp.zeros_like(pool_ref)

    onehot = (lax.broadcasted_iota(jnp.int32, (NG, BM), 0) == b_ref[0]).astype(jnp.float32)
    pool_ref[...] += jnp.dot(onehot, h_ref[...], preferred_element_type=jnp.float32,
                             precision=lax.Precision.HIGHEST)

    @pl.when(m == _MB - 1)
    def _():
        out = jax.nn.relu(pool_ref[...])
        for _ in range(2):
            out = jax.nn.relu(_gru_math(out, out, wih, whh, bih, bhh))
        for W, b, g, be in ((W1, b1, g1, be1), (W2, b2, g2, be2)):
            z = jnp.dot(out, W[...], preferred_element_type=jnp.float32) + b[...]
            mu = jnp.mean(z, axis=-1, keepdims=True)
            var = jnp.mean((z - mu) ** 2, axis=-1, keepdims=True)
            z = (z - mu) / jnp.sqrt(var + 1e-5) * g[...] + be[...]
            out = jax.nn.relu(z)
        emb = jnp.dot(out, We[...], preferred_element_type=jnp.float32) + bee[...]
        o_ref[...] = jnp.dot(emb, Wo[...], preferred_element_type=jnp.float32) + bo[...]


def _head_call(h, batch3d, params):
    l0, l1 = params["lin"]
    gp = params["mol_gru"]
    nt = params["out"]["W"].shape[1]
    d1 = l0["lin"]["W"].shape[1]
    d2 = l1["lin"]["W"].shape[1]

    def full2d(r, c):
        return pl.BlockSpec((r, c), lambda m: (0, 0))

    return pl.pallas_call(
        _head_body,
        grid=(_MB,),
        in_specs=[
            pl.BlockSpec((BM, D), lambda m: (m, 0)),
            pl.BlockSpec((1, 1, BM), lambda m: (m, 0, 0)),
            full2d(D, 3 * D), full2d(D, 3 * D), full2d(1, 3 * D), full2d(1, 3 * D),
            full2d(D, d1), full2d(1, d1), full2d(1, d1), full2d(1, d1),
            full2d(d1, d2), full2d(1, d2), full2d(1, d2), full2d(1, d2),
            full2d(d2, d2), full2d(1, d2),
            full2d(d2, nt), full2d(1, nt),
        ],
        out_specs=pl.BlockSpec((NG, nt), lambda m: (0, 0)),
        out_shape=jax.ShapeDtypeStruct((NG, nt), jnp.float32),
        scratch_shapes=[pltpu.VMEM((NG, D), jnp.float32)],
    )(h, batch3d,
      gp["W_ih"], gp["W_hh"], gp["b_ih"].reshape(1, -1), gp["b_hh"].reshape(1, -1),
      l0["lin"]["W"], l0["lin"]["b"].reshape(1, -1),
      l0["gamma"].reshape(1, -1), l0["beta"].reshape(1, -1),
      l1["lin"]["W"], l1["lin"]["b"].reshape(1, -1),
      l1["gamma"].reshape(1, -1), l1["beta"].reshape(1, -1),
      params["emb"]["W"], params["emb"]["b"].reshape(1, -1),
      params["out"]["W"], params["out"]["b"].reshape(1, -1))


def kernel(x, edge_index, edge_attr, batch, params):
    src = edge_index[0].astype(jnp.int32)
    dst = edge_index[1].astype(jnp.int32)
    batch3d = batch.astype(jnp.int32).reshape(_MB, 1, BM)

    h = x
    for lp in params["agg"]:
        ad2 = lp["a_dst"].reshape(1, -1)
        hn, hn2, s2d = _hn_call(h, lp["node"]["W"], lp["node"]["b"].reshape(1, -1), ad2)
        sdst = s2d[:, 0]
        e = _e_call(edge_attr, lp["edge"]["W"], lp["edge"]["b"].reshape(1, -1))
        num, den = _sc_edge_call(hn2, e, src, dst, sdst, lp["a_src"])
        h = _gru_call(num, den, hn, lp["gru"])

    return _head_call(h, batch3d, params)


# speedup vs baseline: 3.5139x; 1.0014x over previous
import jax
import jax.numpy as jnp
from jax import lax
from jax.experimental import pallas as pl
from jax.experimental.pallas import tpu as pltpu
from jax.experimental.pallas import tpu_sc as plsc

N = 10000
E = 160000
D = 256
NG = 64

NC = 2
NS = 16
L = 16

CH = 32
NCHUNK_BASE = 312
N_PAD = 10016
ROWS_PER_SUB = N_PAD // NS

_MB = 5
BM = N // _MB
_EB = 80
BE = E // _EB


def _split128(z):
    return jnp.stack([z[:, :128], z[:, 128:]], axis=1).reshape(-1, 128)


def _hn_body(h_ref, w_ref, b_ref, ad_ref, hn_ref, hn2_ref, s_ref):
    hn = jnp.dot(h_ref[...], w_ref[...], preferred_element_type=jnp.float32)
    hn = hn + b_ref[...]
    hn_ref[...] = hn
    hn2_ref[...] = _split128(hn)
    s = jnp.sum(hn * ad_ref[0:1, :], axis=-1, keepdims=True)
    s_ref[...] = jnp.broadcast_to(s, s_ref.shape)


def _hn_call(h, W, b, ad2):
    return pl.pallas_call(
        _hn_body,
        grid=(_MB,),
        in_specs=[
            pl.BlockSpec((BM, D), lambda m: (m, 0)),
            pl.BlockSpec((D, D), lambda m: (0, 0)),
            pl.BlockSpec((1, D), lambda m: (0, 0)),
            pl.BlockSpec((1, D), lambda m: (0, 0)),
        ],
        out_specs=[
            pl.BlockSpec((BM, D), lambda m: (m, 0)),
            pl.BlockSpec((2 * BM, 128), lambda m: (m, 0)),
            pl.BlockSpec((BM, 128), lambda m: (m, 0)),
        ],
        out_shape=[
            jax.ShapeDtypeStruct((N, D), jnp.float32),
            jax.ShapeDtypeStruct((2 * N, 128), jnp.float32),
            jax.ShapeDtypeStruct((N, 128), jnp.float32),
        ],
    )(h, W, b, ad2)


def _e_body(ea_ref, w_ref, b_ref, e_ref):
    e = jnp.dot(ea_ref[...], w_ref[...], preferred_element_type=jnp.float32)
    e_ref[...] = _split128(e + b_ref[...])


def _e_call(edge_attr, W, b):
    de = edge_attr.shape[1]
    return pl.pallas_call(
        _e_body,
        grid=(_EB,),
        in_specs=[
            pl.BlockSpec((BE, de), lambda m: (m, 0)),
            pl.BlockSpec((de, D), lambda m: (0, 0)),
            pl.BlockSpec((1, D), lambda m: (0, 0)),
        ],
        out_specs=pl.BlockSpec((2 * BE, 128), lambda m: (m, 0)),
        out_shape=jax.ShapeDtypeStruct((2 * E, 128), jnp.float32),
    )(edge_attr, W, b)


def _sc_edge_body(hn, e, src, dst, sdst, asrc, num, den,
                  rows_v0, rows_v1, e_v0, e_v1, out_v, den_v,
                  src_v0, src_v1, src2_v0, src2_v1, dst_v0, dst_v1,
                  sd_v0, sd_v1,
                  asrc_v, dots_v, w_v, acc, dacc,
                  sem_i0, sem_i1, sem_r0, sem_r1, sem_e0, sem_e1,
                  sem_s0, sem_s1):
    c = lax.axis_index("c")
    s = lax.axis_index("s")

    rows_b = (rows_v0, rows_v1)
    e_b = (e_v0, e_v1)
    src_b = (src_v0, src_v1)
    src2_b = (src2_v0, src2_v1)
    dst_b = (dst_v0, dst_v1)
    sd_b = (sd_v0, sd_v1)
    sem_i = (sem_i0, sem_i1)
    sem_r = (sem_r0, sem_r1)
    sem_e = (sem_e0, sem_e1)
    sem_s = (sem_s0, sem_s1)

    pltpu.sync_copy(asrc, asrc_v)

    zero = jnp.zeros((L,), jnp.float32)

    def _zrow(i, carry):
        for j in range(128 // L):
            out_v[i, pl.ds(j * L, L)] = zero
        den_v[i, :] = zero
        return carry

    lax.fori_loop(0, CH, _zrow, 0)
    nfull = ROWS_PER_SUB // CH
    for kk in range(nfull):
        r0 = s * ROWS_PER_SUB + kk * CH
        pltpu.sync_copy(out_v, acc.at[pl.ds(r0, CH)])
        pltpu.sync_copy(den_v, dacc.at[pl.ds(r0, CH)])
    rem = ROWS_PER_SUB - nfull * CH
    if rem:
        r0 = s * ROWS_PER_SUB + nfull * CH
        pltpu.sync_copy(out_v.at[pl.ds(0, rem)], acc.at[pl.ds(r0, rem)])
        pltpu.sync_copy(den_v.at[pl.ds(0, rem)], dacc.at[pl.ds(r0, rem)])
    plsc.subcore_barrier()

    asrc_vals = [asrc_v[pl.ds(L * j, L)] for j in range(D // L)]
    iota_l = lax.iota(jnp.int32, L) * L
    halfoff = c
    nchunk = NCHUNK_BASE + jnp.where(s < 8, 1, 0)
    base0 = (s * NCHUNK_BASE + jnp.minimum(s, 8)) * CH

    def idx_copy_start(k, b):
        base = base0 + k * CH
        pltpu.async_copy(src.at[pl.ds(base, CH)], src_b[b], sem_i[b])
        pltpu.async_copy(dst.at[pl.ds(base, CH)], dst_b[b], sem_i[b])

    def idx_copy_wait(k, b):
        base = base0 + k * CH
        pltpu.make_async_copy(src.at[pl.ds(base, CH)], src_b[b], sem_i[b]).wait()
        pltpu.make_async_copy(dst.at[pl.ds(base, CH)], dst_b[b], sem_i[b]).wait()

    iota2 = lax.iota(jnp.int32, L) * 2

    def gather_start(k, b):
        base = base0 + k * CH
        for g in range(CH // L):
            sv = src_b[b][pl.ds(g * L, L)] * 2
            plsc.store_scatter(src2_b[b], [iota2 + (2 * L * g)], sv)
            plsc.store_scatter(src2_b[b], [iota2 + (2 * L * g + 1)], sv + 1)
        pltpu.async_copy(hn.at[src2_b[b]], rows_b[b], sem_r[b])
        pltpu.async_copy(e.at[pl.ds(2 * base, 2 * CH)], e_b[b], sem_e[b])
        pltpu.async_copy(sdst.at[dst_b[b]], sd_b[b], sem_s[b])

    def gather_wait(k, b):
        base = base0 + k * CH
        pltpu.make_async_copy(hn.at[src2_b[b]], rows_b[b], sem_r[b]).wait()
        pltpu.make_async_copy(e.at[pl.ds(2 * base, 2 * CH)], e_b[b], sem_e[b]).wait()
        pltpu.make_async_copy(sdst.at[dst_b[b]], sd_b[b], sem_s[b]).wait()

    def compute_chunk(b):
        rows_v = rows_b[b]
        e_v = e_b[b]
        sd_v = sd_b[b]

        @plsc.parallel_loop(0, CH // L, unroll=1)
        def _(g):
            eb = g * L
            db = g * (L * L)
            for i in range(L):
                acc_i = zero
                for j in range(D // L):
                    r = 2 * (eb + i) + j // 8
                    hv = rows_v[r, pl.ds(L * (j % 8), L)]
                    ev = e_v[r, pl.ds(L * (j % 8), L)]
                    m = hv + ev
                    m = jnp.maximum(m, 0.2 * m)
                    acc_i = acc_i + m * asrc_vals[j]
                dots_v[pl.ds(db + i * L, L)] = acc_i
            tot = zero
            for cc in range(L):
                tot = tot + plsc.load_gather(dots_v, [db + iota_l + cc])
            sd = sd_v[pl.ds(eb, L)]
            logit = tot + sd
            logit = jnp.maximum(logit, 0.2 * logit)
            w_v[pl.ds(eb, L)] = jnp.exp(logit)

        @plsc.parallel_loop(0, CH, unroll=4)
        def _(i):
            wi = plsc.load_gather(w_v, [jnp.zeros((L,), jnp.int32) + i])
            r = 2 * i + halfoff
            for j in range(128 // L):
                hv = rows_v[r, pl.ds(L * j, L)]
                ev = e_v[r, pl.ds(L * j, L)]
                m = hv + ev
                m = jnp.maximum(m, 0.2 * m)
                out_v[i, pl.ds(L * j, L)] = m * wi
            den_v[i, :] = wi

    idx_copy_start(0, 0)
    idx_copy_wait(0, 0)
    gather_start(0, 0)
    idx_copy_start(1, 1)

    def chunk_pair(t, carry):
        for b in range(2):
            k = 2 * t + b
            nb = 1 - b

            @pl.when(k + 1 < nchunk)
            def _():
                idx_copy_wait(k + 1, nb)
                gather_start(k + 1, nb)

            @pl.when(k < nchunk)
            def _():
                gather_wait(k, b)
                compute_chunk(b)
                pltpu.sync_copy(out_v, acc.at[dst_b[b]], add=True)
                pltpu.sync_copy(den_v, dacc.at[dst_b[b]], add=True)

            @pl.when(k + 2 < nchunk)
            def _():
                idx_copy_start(k + 2, b)
        return carry

    lax.fori_loop(0, (NCHUNK_BASE + 2) // 2, chunk_pair, 0)
    plsc.subcore_barrier()

    r0 = s * ROWS_PER_SUB
    pltpu.sync_copy(acc.at[pl.ds(r0, ROWS_PER_SUB)],
                    num.at[c, pl.ds(r0, ROWS_PER_SUB)])
    pltpu.sync_copy(dacc.at[pl.ds(r0, ROWS_PER_SUB)],
                    den.at[c, pl.ds(r0, ROWS_PER_SUB)])


def _sc_edge_call(hn_b, e_b, src, dst, sdst, asrc):
    mesh = plsc.VectorSubcoreMesh(core_axis_name="c", subcore_axis_name="s")
    f = pl.kernel(
        _sc_edge_body,
        mesh=mesh,
        compiler_params=pltpu.CompilerParams(
            use_tc_tiling_on_sc=False, needs_layout_passes=False),
        out_type=[
            jax.ShapeDtypeStruct((NC, N_PAD, 128), jnp.float32),
            jax.ShapeDtypeStruct((NC, N_PAD, L), jnp.float32),
        ],
        scratch_types=[
            pltpu.VMEM((2 * CH, 128), jnp.float32),
            pltpu.VMEM((2 * CH, 128), jnp.float32),
            pltpu.VMEM((2 * CH, 128), jnp.float32),
            pltpu.VMEM((2 * CH, 128), jnp.float32),
            pltpu.VMEM((CH, 128), jnp.float32),
            pltpu.VMEM((CH, L), jnp.float32),
            pltpu.VMEM((CH,), jnp.int32),
            pltpu.VMEM((CH,), jnp.int32),
            pltpu.VMEM((2 * CH,), jnp.int32),
            pltpu.VMEM((2 * CH,), jnp.int32),
            pltpu.VMEM((CH,), jnp.int32),
            pltpu.VMEM((CH,), jnp.int32),
            pltpu.VMEM((CH,), jnp.float32),
            pltpu.VMEM((CH,), jnp.float32),
            pltpu.VMEM((D,), jnp.float32),
            pltpu.VMEM((CH * L,), jnp.float32),
            pltpu.VMEM((CH,), jnp.float32),
            pltpu.VMEM_SHARED((N_PAD, 128), jnp.float32),
            pltpu.VMEM_SHARED((N_PAD, L), jnp.float32),
            pltpu.SemaphoreType.DMA,
            pltpu.SemaphoreType.DMA,
            pltpu.SemaphoreType.DMA,
            pltpu.SemaphoreType.DMA,
            pltpu.SemaphoreType.DMA,
            pltpu.SemaphoreType.DMA,
            pltpu.SemaphoreType.DMA,
            pltpu.SemaphoreType.DMA,
        ],
    )
    return f(hn_b, e_b, src, dst, sdst, asrc)


def _gru_math(x, h, wih_ref, whh_ref, bih_ref, bhh_ref):
    gi = jnp.dot(x, wih_ref[...], preferred_element_type=jnp.float32) + bih_ref[...]
    gh = jnp.dot(h, whh_ref[...], preferred_element_type=jnp.float32) + bhh_ref[...]
    dh = h.shape[-1]
    r = jax.nn.sigmoid(gi[:, :dh] + gh[:, :dh])
    z = jax.nn.sigmoid(gi[:, dh:2 * dh] + gh[:, dh:2 * dh])
    n = jnp.tanh(gi[:, 2 * dh:] + r * gh[:, 2 * dh:])
    return (1.0 - z) * n + z * h


def _gru_body(a0_ref, a1_ref, dn_ref, hn_ref, wih_ref, whh_ref, bih_ref,
              bhh_ref, h_ref):
    den = dn_ref[0][:, 0:1] + 1e-16
    num = jnp.concatenate([a0_ref[0], a1_ref[0]], axis=1)
    agg = num / den
    hn = hn_ref[...]
    h_ref[...] = jax.nn.relu(_gru_math(agg, hn, wih_ref, whh_ref, bih_ref, bhh_ref))


def _gru_call(num, den, hn, gp):
    return pl.pallas_call(
        _gru_body,
        grid=(_MB,),
        in_specs=[
            pl.BlockSpec((1, BM, 128), lambda m: (0, m, 0)),
            pl.BlockSpec((1, BM, 128), lambda m: (1, m, 0)),
            pl.BlockSpec((1, BM, L), lambda m: (0, m, 0)),
            pl.BlockSpec((BM, D), lambda m: (m, 0)),
            pl.BlockSpec((D, 3 * D), lambda m: (0, 0)),
            pl.BlockSpec((D, 3 * D), lambda m: (0, 0)),
            pl.BlockSpec((1, 3 * D), lambda m: (0, 0)),
            pl.BlockSpec((1, 3 * D), lambda m: (0, 0)),
        ],
        out_specs=pl.BlockSpec((BM, D), lambda m: (m, 0)),
        out_shape=jax.ShapeDtypeStruct((N, D), jnp.float32),
    )(num, num, den, hn, gp["W_ih"], gp["W_hh"],
      gp["b_ih"].reshape(1, -1), gp["b_hh"].reshape(1, -1))


def _head_body(h_ref, b_ref, wih, whh, bih, bhh,
               W1, b1, g1, be1, W2, b2, g2, be2, We, bee, Wo, bo,
               o_ref, pool_ref):
    m = pl.program_id(0)

    @pl.when(m == 0)
    def _():
        pool_ref[...] = jnp.zeros_like(pool_ref)

    onehot = (lax.broadcasted_iota(jnp.int32, (NG, BM), 0) == b_ref[0]).astype(jnp.float32)
    pool_ref[...] += jnp.dot(onehot, h_ref[...], preferred_element_type=jnp.float32,
                             precision=lax.Precision.HIGHEST)

    @pl.when(m == _MB - 1)
    def _():
        out = jax.nn.relu(pool_ref[...])
        for _ in range(2):
            out = jax.nn.relu(_gru_math(out, out, wih, whh, bih, bhh))
        for W, b, g, be in ((W1, b1, g1, be1), (W2, b2, g2, be2)):
            z = jnp.dot(out, W[...], preferred_element_type=jnp.float32) + b[...]
            mu = jnp.mean(z, axis=-1, keepdims=True)
            var = jnp.mean((z - mu) ** 2, axis=-1, keepdims=True)
            z = (z - mu) / jnp.sqrt(var + 1e-5) * g[...] + be[...]
            out = jax.nn.relu(z)
        emb = jnp.dot(out, We[...], preferred_element_type=jnp.float32) + bee[...]
        o_ref[...] = jnp.dot(emb, Wo[...], preferred_element_type=jnp.float32) + bo[...]


def _head_call(h, batch3d, params):
    l0, l1 = params["lin"]
    gp = params["mol_gru"]
    nt = params["out"]["W"].shape[1]
    d1 = l0["lin"]["W"].shape[1]
    d2 = l1["lin"]["W"].shape[1]

    def full2d(r, c):
        return pl.BlockSpec((r, c), lambda m: (0, 0))

    return pl.pallas_call(
        _head_body,
        grid=(_MB,),
        in_specs=[
            pl.BlockSpec((BM, D), lambda m: (m, 0)),
            pl.BlockSpec((1, 1, BM), lambda m: (m, 0, 0)),
            full2d(D, 3 * D), full2d(D, 3 * D), full2d(1, 3 * D), full2d(1, 3 * D),
            full2d(D, d1), full2d(1, d1), full2d(1, d1), full2d(1, d1),
            full2d(d1, d2), full2d(1, d2), full2d(1, d2), full2d(1, d2),
            full2d(d2, d2), full2d(1, d2),
            full2d(d2, nt), full2d(1, nt),
        ],
        out_specs=pl.BlockSpec((NG, nt), lambda m: (0, 0)),
        out_shape=jax.ShapeDtypeStruct((NG, nt), jnp.float32),
        scratch_shapes=[pltpu.VMEM((NG, D), jnp.float32)],
    )(h, batch3d,
      gp["W_ih"], gp["W_hh"], gp["b_ih"].reshape(1, -1), gp["b_hh"].reshape(1, -1),
      l0["lin"]["W"], l0["lin"]["b"].reshape(1, -1),
      l0["gamma"].reshape(1, -1), l0["beta"].reshape(1, -1),
      l1["lin"]["W"], l1["lin"]["b"].reshape(1, -1),
      l1["gamma"].reshape(1, -1), l1["beta"].reshape(1, -1),
      params["emb"]["W"], params["emb"]["b"].reshape(1, -1),
      params["out"]["W"], params["out"]["b"].reshape(1, -1))


def kernel(x, edge_index, edge_attr, batch, params):
    src = edge_index[0].astype(jnp.int32)
    dst = edge_index[1].astype(jnp.int32)
    batch3d = batch.astype(jnp.int32).reshape(_MB, 1, BM)

    h = x
    for lp in params["agg"]:
        ad2 = lp["a_dst"].reshape(1, -1)
        hn, hn2, s2d = _hn_call(h, lp["node"]["W"], lp["node"]["b"].reshape(1, -1), ad2)
        sdst = s2d[:, 0]
        e = _e_call(edge_attr, lp["edge"]["W"], lp["edge"]["b"].reshape(1, -1))
        num, den = _sc_edge_call(hn2, e, src, dst, sdst, lp["a_src"])
        h = _gru_call(num, den, hn, lp["gru"])

    return _head_call(h, batch3d, params)
